# Initial kernel scaffold; baseline (speedup 1.0000x reference)
import jax
import jax.numpy as jnp
from jax.experimental import pallas as pl


def kernel(h, e, edge_index, W_h, W_e, W_proj, b_proj, W_attn, gamma_h, beta_h, gamma_e, beta_e):
    raise NotImplementedError("write your pallas kernel here")


# speedup vs baseline: 15.2058x; 15.2058x over previous
import functools

import jax
import jax.numpy as jnp
from jax import lax
from jax.experimental import pallas as pl
from jax.experimental.pallas import tpu as pltpu
from jax.experimental.pallas import tpu_sc as plsc

N_NODES = 10000
N_EDGES = 320000
IN_DIM = 128
OUT_DIM = 32
N_HEADS = 4
HD = N_HEADS * OUT_DIM
EPS = 1e-5

_NC = 2
_NS = 16
_NW = _NC * _NS
_CH = 128
_NCHT = N_EDGES // _CH
_CPW = _NCHT // _NW
_NREM = _NCHT - _CPW * _NW
_NPAD = 10240
_SR = _NPAD // _NS

_BE = 3200
_NBE = N_EDGES // _BE


def _stage_a_body(h_ref, wh_ref, wa_ref, zh_ref, dt_ref):
    h = h_ref[...]
    wa = wa_ref[...]
    zs, drows = [], []
    for j in range(N_HEADS):
        z = jnp.dot(h, wh_ref[j])
        zs.append(z)
        wd = wa[j, 2 * OUT_DIM:3 * OUT_DIM][None, :]
        drows.append(lax.dot_general(wd, z, (((1,), (1,)), ((), ()))))
    zh_ref[...] = jnp.concatenate(zs, axis=1)
    dt_ref[...] = jnp.concatenate(drows, axis=0)


def _stage_a(h, w_h, w_attn):
    return pl.pallas_call(
        _stage_a_body,
        grid=(1,),
        in_specs=[
            pl.BlockSpec((N_NODES, IN_DIM), lambda i: (0, 0)),
            pl.BlockSpec((N_HEADS, IN_DIM, OUT_DIM), lambda i: (0, 0, 0)),
            pl.BlockSpec((N_HEADS, 3 * OUT_DIM), lambda i: (0, 0)),
        ],
        out_specs=[
            pl.BlockSpec((N_NODES, HD), lambda i: (0, 0)),
            pl.BlockSpec((N_HEADS, N_NODES), lambda i: (0, 0)),
        ],
        out_shape=[
            jax.ShapeDtypeStruct((N_NODES, HD), jnp.float32),
            jax.ShapeDtypeStruct((N_HEADS, N_NODES), jnp.float32),
        ],
    )(h, w_h, w_attn)


def _gather_body(zh_hbm, dt0, dt1, dt2, dt3, src_hbm, dst_hbm,
                 gz_hbm, dd0, dd1, dd2, dd3,
                 sidx, didx, rows, dvals):
    wid = lax.axis_index("s") * _NC + lax.axis_index("c")
    dts = [dt0, dt1, dt2, dt3]
    dds = [dd0, dd1, dd2, dd3]

    def step(k, carry):
        off = (k * _NW + wid) * _CH
        pltpu.sync_copy(src_hbm.at[pl.ds(off, _CH)], sidx)
        pltpu.sync_copy(dst_hbm.at[pl.ds(off, _CH)], didx)
        pltpu.sync_copy(zh_hbm.at[sidx], rows)
        pltpu.sync_copy(rows, gz_hbm.at[pl.ds(off, _CH)])
        for j in range(N_HEADS):
            pltpu.sync_copy(dts[j].at[didx], dvals)
            pltpu.sync_copy(dvals, dds[j].at[0, pl.ds(off, _CH)])
        return carry

    niter = _CPW + jnp.where(wid < _NREM, 1, 0)
    lax.fori_loop(0, niter, step, 0)


def _stage_g(zh, dtab_t, src, dst):
    f = functools.partial(
        pl.kernel,
        out_type=(
            jax.ShapeDtypeStruct((N_EDGES, HD), jnp.float32),
        ) + tuple(jax.ShapeDtypeStruct((1, N_EDGES), jnp.float32)
                  for _ in range(N_HEADS)),
        mesh=plsc.VectorSubcoreMesh(core_axis_name="c", subcore_axis_name="s"),
        scratch_types=[
            pltpu.VMEM((_CH,), jnp.int32),
            pltpu.VMEM((_CH,), jnp.int32),
            pltpu.VMEM((_CH, HD), jnp.float32),
            pltpu.VMEM((_CH,), jnp.float32),
        ],
    )(_gather_body)
    dts = [jnp.reshape(dtab_t[j], (N_NODES,)) for j in range(N_HEADS)]
    return f(zh, *dts, src, dst)


def _stage_b_body(e_ref, gz_ref, dd0, dd1, dd2, dd3, we_ref, wa_ref,
                  y_ref, ex0, ex1, ex2, ex3):
    wa = wa_ref[...]
    gz = gz_ref[...]
    ve_cols = [jnp.dot(we_ref[j], wa[j, 0:OUT_DIM][:, None])
               for j in range(N_HEADS)]
    a = jnp.dot(e_ref[...], jnp.concatenate(ve_cols, axis=1))
    s_cols = [jnp.dot(gz[:, OUT_DIM * j:OUT_DIM * (j + 1)],
                      wa[j, OUT_DIM:2 * OUT_DIM][:, None])
              for j in range(N_HEADS)]
    a = a + jnp.concatenate(s_cols, axis=1)
    ones11 = jnp.ones((1, 1), jnp.float32)
    dd_cols = [lax.dot_general(dd[...], ones11, (((0,), (0,)), ((), ())))
               for dd in (dd0, dd1, dd2, dd3)]
    a = a + jnp.concatenate(dd_cols, axis=1)
    a = jnp.where(a > 0, a, 0.01 * a)
    ex = jnp.exp(a)
    col = lax.broadcasted_iota(jnp.int32, (1, N_HEADS), 1)
    for j, exr in enumerate((ex0, ex1, ex2, ex3)):
        onehot = jnp.where(col == j, 1.0, 0.0)
        exr[...] = lax.dot_general(onehot, ex, (((1,), (1,)), ((), ())))
    y_cols = [gz[:, OUT_DIM * j:OUT_DIM * (j + 1)] * ex[:, j:j + 1]
              for j in range(N_HEADS)]
    y_ref[...] = jnp.concatenate(y_cols, axis=1)


def _stage_b(e, gz, dds, w_e, w_attn):
    edge_spec = pl.BlockSpec((1, _BE), lambda i: (0, i))
    return pl.pallas_call(
        _stage_b_body,
        grid=(_NBE,),
        in_specs=[
            pl.BlockSpec((_BE, IN_DIM), lambda i: (i, 0)),
            pl.BlockSpec((_BE, HD), lambda i: (i, 0)),
            edge_spec, edge_spec, edge_spec, edge_spec,
            pl.BlockSpec((N_HEADS, IN_DIM, OUT_DIM), lambda i: (0, 0, 0)),
            pl.BlockSpec((N_HEADS, 3 * OUT_DIM), lambda i: (0, 0)),
        ],
        out_specs=[
            pl.BlockSpec((_BE, HD), lambda i: (i, 0)),
            edge_spec, edge_spec, edge_spec, edge_spec,
        ],
        out_shape=[
            jax.ShapeDtypeStruct((N_EDGES, HD), jnp.float32),
        ] + [jax.ShapeDtypeStruct((1, N_EDGES), jnp.float32)] * N_HEADS,
    )(e, gz, *dds, w_e, w_attn)


def _scatter_body(y_hbm, ex0, ex1, ex2, ex3, dst_hbm, zh_hbm, zd_hbm,
                  ph_hbm, pd0, pd1, pd2, pd3,
                  didx, yrows, exr, hsh, dsh0, dsh1, dsh2, dsh3):
    cid = lax.axis_index("c")
    sid = lax.axis_index("s")
    wid = sid * _NC + cid
    exs = [ex0, ex1, ex2, ex3]
    dshs = [dsh0, dsh1, dsh2, dsh3]
    pds = [pd0, pd1, pd2, pd3]

    pltpu.sync_copy(zh_hbm.at[pl.ds(sid * _SR, _SR)],
                    hsh.at[pl.ds(sid * _SR, _SR)])
    for j in range(N_HEADS):
        pltpu.sync_copy(zd_hbm.at[pl.ds(sid * _SR, _SR)],
                        dshs[j].at[pl.ds(sid * _SR, _SR)])
    plsc.subcore_barrier()

    def step(k, carry):
        off = (k * _NW + wid) * _CH
        pltpu.sync_copy(dst_hbm.at[pl.ds(off, _CH)], didx)
        pltpu.sync_copy(y_hbm.at[pl.ds(off, _CH)], yrows)
        pltpu.sync_copy(yrows, hsh.at[didx], add=True)
        for j in range(N_HEADS):
            pltpu.sync_copy(exs[j].at[0, pl.ds(off, _CH)], exr)
            pltpu.sync_copy(exr, dshs[j].at[didx], add=True)
        return carry

    niter = _CPW + jnp.where(wid < _NREM, 1, 0)
    lax.fori_loop(0, niter, step, 0)
    plsc.subcore_barrier()

    out_off = cid * _NPAD + sid * _SR
    pltpu.sync_copy(hsh.at[pl.ds(sid * _SR, _SR)],
                    ph_hbm.at[pl.ds(out_off, _SR)])
    for j in range(N_HEADS):
        pltpu.sync_copy(dshs[j].at[pl.ds(sid * _SR, _SR)],
                        pds[j].at[pl.ds(out_off, _SR)])


def _stage_s(y, exs, dst, zeros_h, zeros_d):
    f = functools.partial(
        pl.kernel,
        out_type=(
            jax.ShapeDtypeStruct((_NC * _NPAD, HD), jnp.float32),
        ) + tuple(jax.ShapeDtypeStruct((_NC * _NPAD,), jnp.float32)
                  for _ in range(N_HEADS)),
        mesh=plsc.VectorSubcoreMesh(core_axis_name="c", subcore_axis_name="s"),
        scratch_types=[
            pltpu.VMEM((_CH,), jnp.int32),
            pltpu.VMEM((_CH, HD), jnp.float32),
            pltpu.VMEM((_CH,), jnp.float32),
            pltpu.VMEM_SHARED((_NPAD, HD), jnp.float32),
        ] + [pltpu.VMEM_SHARED((_NPAD,), jnp.float32)] * N_HEADS,
    )(_scatter_body)
    return f(y, *exs, dst, zeros_h, zeros_d)


def _stage_d_body(ph_ref, pd0, pd1, pd2, pd3, g_ref, b_ref, out_ref):
    hagg = (ph_ref[0:N_NODES, :]
            + ph_ref[_NPAD:_NPAD + N_NODES, :])
    acc = jnp.zeros((N_NODES, OUT_DIM), dtype=jnp.float32)
    inv_n = 1.0 / N_NODES
    ones11 = jnp.ones((1, 1), jnp.float32)
    for j, pd in enumerate((pd0, pd1, pd2, pd3)):
        den_row = pd[0:1, :] + pd[1:2, :]
        dj = lax.dot_general(den_row, ones11, (((0,), (0,)), ((), ())))
        dj = dj[0:N_NODES, :]
        x = hagg[:, OUT_DIM * j:OUT_DIM * (j + 1)]
        x = jnp.where(dj > 0, x / dj, 0.0)
        mu = jnp.sum(x, axis=0, keepdims=True) * inv_n
        xc = x - mu
        var = jnp.sum(xc * xc, axis=0, keepdims=True) * inv_n
        y = xc * lax.rsqrt(var + EPS) * g_ref[j][None, :] + b_ref[j][None, :]
        y = jnp.where(y > 0, y, jnp.exp(jnp.minimum(y, 0.0)) - 1.0)
        acc = acc + y
    out_ref[...] = acc


def _stage_d(ph, pds, gamma_h, beta_h):
    pd_spec = pl.BlockSpec((_NC, _NPAD), lambda i: (0, 0))
    return pl.pallas_call(
        _stage_d_body,
        grid=(1,),
        in_specs=[
            pl.BlockSpec((_NC * _NPAD, HD), lambda i: (0, 0)),
            pd_spec, pd_spec, pd_spec, pd_spec,
            pl.BlockSpec((N_HEADS, OUT_DIM), lambda i: (0, 0)),
            pl.BlockSpec((N_HEADS, OUT_DIM), lambda i: (0, 0)),
        ],
        out_specs=pl.BlockSpec((N_NODES, OUT_DIM), lambda i: (0, 0)),
        out_shape=jax.ShapeDtypeStruct((N_NODES, OUT_DIM), jnp.float32),
    )(ph, *pds, gamma_h, beta_h)


def kernel(h, e, edge_index, W_h, W_e, W_proj, b_proj, W_attn,
           gamma_h, beta_h, gamma_e, beta_e):
    src = edge_index[0].astype(jnp.int32)
    dst = edge_index[1].astype(jnp.int32)

    zh, dtab_t = _stage_a(h, W_h, W_attn)
    gz, *dds = _stage_g(zh, dtab_t, src, dst)
    y, *exs = _stage_b(e, gz, dds, W_e, W_attn)
    zeros_h = jnp.zeros((_NPAD, HD), jnp.float32)
    zeros_d = jnp.zeros((_NPAD,), jnp.float32)
    ph, *pds = _stage_s(y, exs, dst, zeros_h, zeros_d)
    pds2 = [jnp.reshape(p, (_NC, _NPAD)) for p in pds]
    h_out = _stage_d(ph, pds2, gamma_h, beta_h)
    return (h_out, e)


# speedup vs baseline: 25.5456x; 1.6800x over previous
import functools

import jax
import jax.numpy as jnp
from jax import lax
from jax.experimental import pallas as pl
from jax.experimental.pallas import tpu as pltpu
from jax.experimental.pallas import tpu_sc as plsc

N_NODES = 10000
N_EDGES = 320000
IN_DIM = 128
OUT_DIM = 32
N_HEADS = 4
HD = N_HEADS * OUT_DIM
EPS = 1e-5

_NC = 2
_NS = 16
_NW = _NC * _NS
_CH = 128
_SCE = 512
_NSC = N_EDGES // _SCE
_SPW = _NSC // _NW
_NREM = _NSC - _SPW * _NW
_SCS = 256
_NSCS = N_EDGES // _SCS
_SPWS = _NSCS // _NW
_NREMS = _NSCS - _SPWS * _NW
_NPAD = 10240
_SR = _NPAD // _NS

_BE = 3200
_NBE = N_EDGES // _BE
_BE2 = 6400
_NBE2 = N_EDGES // _BE2


def _stage_a1_body(h_ref, wh_ref, wa_ref, zh_ref, dt_ref):
    wa = wa_ref[...]
    h = h_ref[...]
    zs, drows = [], []
    for j in range(N_HEADS):
        z = jnp.dot(h, wh_ref[j])
        zs.append(z)
        wd = wa[j, 2 * OUT_DIM:3 * OUT_DIM][None, :]
        drows.append(lax.dot_general(wd, z, (((1,), (1,)), ((), ()))))
    zh_ref[...] = jnp.concatenate(zs, axis=1)
    dt_ref[...] = jnp.concatenate(drows, axis=0)


def _stage_a1(h, w_h, w_attn):
    return pl.pallas_call(
        _stage_a1_body,
        grid=(1,),
        in_specs=[
            pl.BlockSpec((N_NODES, IN_DIM), lambda i: (0, 0)),
            pl.BlockSpec((N_HEADS, IN_DIM, OUT_DIM), lambda i: (0, 0, 0)),
            pl.BlockSpec((N_HEADS, 3 * OUT_DIM), lambda i: (0, 0)),
        ],
        out_specs=[
            pl.BlockSpec((N_NODES, HD), lambda i: (0, 0)),
            pl.BlockSpec((N_HEADS, N_NODES), lambda i: (0, 0)),
        ],
        out_shape=[
            jax.ShapeDtypeStruct((N_NODES, HD), jnp.float32),
            jax.ShapeDtypeStruct((N_HEADS, N_NODES), jnp.float32),
        ],
    )(h, w_h, w_attn)


def _stage_a2_body(e_ref, wa_ref, we_ref, ae_ref):
    wa = wa_ref[...]
    ve_cols = [jnp.dot(we_ref[j], wa[j, 0:OUT_DIM][:, None])
               for j in range(N_HEADS)]
    v_e = jnp.concatenate(ve_cols, axis=1)
    ae_ref[...] = lax.dot_general(v_e, e_ref[...],
                                  (((0,), (1,)), ((), ())))


def _stage_a2(e, w_attn, w_e):
    return pl.pallas_call(
        _stage_a2_body,
        grid=(_NBE,),
        in_specs=[
            pl.BlockSpec((_BE, IN_DIM), lambda i: (i, 0)),
            pl.BlockSpec((N_HEADS, 3 * OUT_DIM), lambda i: (0, 0)),
            pl.BlockSpec((N_HEADS, IN_DIM, OUT_DIM), lambda i: (0, 0, 0)),
        ],
        out_specs=pl.BlockSpec((N_HEADS, _BE), lambda i: (0, i)),
        out_shape=jax.ShapeDtypeStruct((N_HEADS, N_EDGES), jnp.float32),
    )(e, w_attn, w_e)


def _gather_body(zh_hbm, dt0, dt1, dt2, dt3, src_hbm, dst_hbm,
                 gz_hbm, ddt_hbm,
                 sidx, didx, rows, ddbuf, sem, semd):
    wid = lax.axis_index("s") * _NC + lax.axis_index("c")
    dts = [dt0, dt1, dt2, dt3]

    def step(k, carry):
        off = (k * _NW + wid) * _SCE
        pltpu.sync_copy(src_hbm.at[pl.ds(off, _SCE)], sidx)
        pltpu.sync_copy(dst_hbm.at[pl.ds(off, _SCE)], didx)
        handles = [
            pltpu.async_copy(
                zh_hbm.at[sidx.at[pl.ds(t * _CH, _CH)]],
                rows.at[pl.ds(t * _CH, _CH)], sem)
            for t in range(_SCE // _CH)
        ]
        dhandles = [
            pltpu.async_copy(
                dts[j].at[didx.at[pl.ds(t * _CH, _CH)]],
                ddbuf.at[j, pl.ds(t * _CH, _CH)], semd)
            for t in range(_SCE // _CH)
            for j in range(N_HEADS)
        ]
        for hdl in handles:
            hdl.wait()
        pltpu.sync_copy(rows, gz_hbm.at[pl.ds(off, _SCE)])
        for hdl in dhandles:
            hdl.wait()
        pltpu.sync_copy(ddbuf, ddt_hbm.at[:, pl.ds(off, _SCE)])
        return carry

    niter = _SPW + jnp.where(wid < _NREM, 1, 0)
    lax.fori_loop(0, niter, step, 0)


def _stage_g(zh, dtab_t, src, dst):
    f = functools.partial(
        pl.kernel,
        out_type=(
            jax.ShapeDtypeStruct((N_EDGES, HD), jnp.float32),
            jax.ShapeDtypeStruct((N_HEADS, N_EDGES), jnp.float32),
        ),
        mesh=plsc.VectorSubcoreMesh(core_axis_name="c", subcore_axis_name="s"),
        scratch_types=[
            pltpu.VMEM((_SCE,), jnp.int32),
            pltpu.VMEM((_SCE,), jnp.int32),
            pltpu.VMEM((_SCE, HD), jnp.float32),
            pltpu.VMEM((N_HEADS, _SCE), jnp.float32),
            pltpu.SemaphoreType.DMA,
            pltpu.SemaphoreType.DMA,
        ],
    )(_gather_body)
    dts = [jnp.reshape(dtab_t[j], (N_NODES,)) for j in range(N_HEADS)]
    return f(zh, *dts, src, dst)


def _stage_b_body(gz_ref, ae_ref, dd_ref, wa_ref, y_ref, ex_ref):
    wa = wa_ref[...]
    gz = gz_ref[...]
    s_cols = [jnp.dot(gz[:, OUT_DIM * j:OUT_DIM * (j + 1)],
                      wa[j, OUT_DIM:2 * OUT_DIM][:, None])
              for j in range(N_HEADS)]
    eye4 = jnp.where(
        lax.broadcasted_iota(jnp.int32, (N_HEADS, N_HEADS), 0)
        == lax.broadcasted_iota(jnp.int32, (N_HEADS, N_HEADS), 1), 1.0, 0.0)
    add_t = ae_ref[...] + dd_ref[...]
    a = lax.dot_general(add_t, eye4, (((0,), (0,)), ((), ())))
    a = a + jnp.concatenate(s_cols, axis=1)
    a = jnp.where(a > 0, a, 0.01 * a)
    ex = jnp.exp(a)
    ex_ref[...] = lax.dot_general(eye4, ex, (((1,), (1,)), ((), ())))
    y_cols = [gz[:, OUT_DIM * j:OUT_DIM * (j + 1)] * ex[:, j:j + 1]
              for j in range(N_HEADS)]
    y_ref[...] = jnp.concatenate(y_cols, axis=1)


def _stage_b(gz, ae_t, dd_t, w_attn):
    edge_spec = pl.BlockSpec((N_HEADS, _BE2), lambda i: (0, i))
    return pl.pallas_call(
        _stage_b_body,
        grid=(_NBE2,),
        in_specs=[
            pl.BlockSpec((_BE2, HD), lambda i: (i, 0)),
            edge_spec,
            edge_spec,
            pl.BlockSpec((N_HEADS, 3 * OUT_DIM), lambda i: (0, 0)),
        ],
        out_specs=[
            pl.BlockSpec((_BE2, HD), lambda i: (i, 0)),
            edge_spec,
        ],
        out_shape=[
            jax.ShapeDtypeStruct((N_EDGES, HD), jnp.float32),
            jax.ShapeDtypeStruct((N_HEADS, N_EDGES), jnp.float32),
        ],
    )(gz, ae_t, dd_t, w_attn)


def _scatter_body(y_hbm, ex_hbm, dst_hbm, zh_hbm, zd_hbm,
                  ph_hbm, pd0, pd1, pd2, pd3,
                  didx0, didx1, yrows, exbuf,
                  hsh, dsh0, dsh1, dsh2, dsh3, semi, sems, semd):
    cid = lax.axis_index("c")
    sid = lax.axis_index("s")
    wid = sid * _NC + cid
    didxs = [didx0, didx1]
    dshs = [dsh0, dsh1, dsh2, dsh3]
    pds = [pd0, pd1, pd2, pd3]

    pltpu.sync_copy(zh_hbm.at[pl.ds(sid * _SR, _SR)],
                    hsh.at[pl.ds(sid * _SR, _SR)])
    for j in range(N_HEADS):
        pltpu.sync_copy(zd_hbm.at[pl.ds(sid * _SR, _SR)],
                        dshs[j].at[pl.ds(sid * _SR, _SR)])
    plsc.subcore_barrier()

    def step(k, carry):
        off = (k * _NW + wid) * _SCS
        ih = [pltpu.async_copy(dst_hbm.at[pl.ds(off + t * _CH, _CH)],
                               didxs[t], semi)
              for t in range(_SCS // _CH)]
        pltpu.sync_copy(y_hbm.at[pl.ds(off, _SCS)], yrows)
        pltpu.sync_copy(ex_hbm.at[:, pl.ds(off, _SCS)], exbuf)
        for h in ih:
            h.wait()
        ah = []
        for t in range(_SCS // _CH):
            ah.append(pltpu.async_copy(
                yrows.at[pl.ds(t * _CH, _CH)], hsh.at[didxs[t]], sems,
                add=True))
            for j in range(N_HEADS):
                ah.append(pltpu.async_copy(
                    exbuf.at[j, pl.ds(t * _CH, _CH)], dshs[j].at[didxs[t]],
                    semd, add=True))
        for h in ah:
            h.wait()
        return carry

    niter = _SPWS + jnp.where(wid < _NREMS, 1, 0)
    lax.fori_loop(0, niter, step, 0)
    plsc.subcore_barrier()

    out_off = cid * _NPAD + sid * _SR
    pltpu.sync_copy(hsh.at[pl.ds(sid * _SR, _SR)],
                    ph_hbm.at[pl.ds(out_off, _SR)])
    for j in range(N_HEADS):
        pltpu.sync_copy(dshs[j].at[pl.ds(sid * _SR, _SR)],
                        pds[j].at[pl.ds(out_off, _SR)])


def _stage_s(y, ex_t, dst, zeros_h, zeros_d):
    f = functools.partial(
        pl.kernel,
        out_type=(
            jax.ShapeDtypeStruct((_NC * _NPAD, HD), jnp.float32),
        ) + tuple(jax.ShapeDtypeStruct((_NC * _NPAD,), jnp.float32)
                  for _ in range(N_HEADS)),
        mesh=plsc.VectorSubcoreMesh(core_axis_name="c", subcore_axis_name="s"),
        scratch_types=[
            pltpu.VMEM((_CH,), jnp.int32),
            pltpu.VMEM((_CH,), jnp.int32),
            pltpu.VMEM((_SCS, HD), jnp.float32),
            pltpu.VMEM((N_HEADS, _SCS), jnp.float32),
            pltpu.VMEM_SHARED((_NPAD, HD), jnp.float32),
        ] + [pltpu.VMEM_SHARED((_NPAD,), jnp.float32)] * N_HEADS
          + [pltpu.SemaphoreType.DMA] * 3,
    )(_scatter_body)
    return f(y, ex_t, dst, zeros_h, zeros_d)


def _stage_d_body(ph_ref, pd0, pd1, pd2, pd3, g_ref, b_ref, out_ref):
    hagg = (ph_ref[0:N_NODES, :]
            + ph_ref[_NPAD:_NPAD + N_NODES, :])
    acc = jnp.zeros((N_NODES, OUT_DIM), dtype=jnp.float32)
    inv_n = 1.0 / N_NODES
    ones11 = jnp.ones((1, 1), jnp.float32)
    for j, pd in enumerate((pd0, pd1, pd2, pd3)):
        den_row = pd[0:1, :] + pd[1:2, :]
        dj = lax.dot_general(den_row, ones11, (((0,), (0,)), ((), ())))
        dj = dj[0:N_NODES, :]
        x = hagg[:, OUT_DIM * j:OUT_DIM * (j + 1)]
        x = jnp.where(dj > 0, x / dj, 0.0)
        mu = jnp.sum(x, axis=0, keepdims=True) * inv_n
        xc = x - mu
        var = jnp.sum(xc * xc, axis=0, keepdims=True) * inv_n
        y = xc * lax.rsqrt(var + EPS) * g_ref[j][None, :] + b_ref[j][None, :]
        y = jnp.where(y > 0, y, jnp.exp(jnp.minimum(y, 0.0)) - 1.0)
        acc = acc + y
    out_ref[...] = acc


def _stage_d(ph, pds, gamma_h, beta_h):
    pd_spec = pl.BlockSpec((_NC, _NPAD), lambda i: (0, 0))
    return pl.pallas_call(
        _stage_d_body,
        grid=(1,),
        in_specs=[
            pl.BlockSpec((_NC * _NPAD, HD), lambda i: (0, 0)),
            pd_spec, pd_spec, pd_spec, pd_spec,
            pl.BlockSpec((N_HEADS, OUT_DIM), lambda i: (0, 0)),
            pl.BlockSpec((N_HEADS, OUT_DIM), lambda i: (0, 0)),
        ],
        out_specs=pl.BlockSpec((N_NODES, OUT_DIM), lambda i: (0, 0)),
        out_shape=jax.ShapeDtypeStruct((N_NODES, OUT_DIM), jnp.float32),
    )(ph, *pds, gamma_h, beta_h)


def kernel(h, e, edge_index, W_h, W_e, W_proj, b_proj, W_attn,
           gamma_h, beta_h, gamma_e, beta_e):
    src = edge_index[0].astype(jnp.int32)
    dst = edge_index[1].astype(jnp.int32)

    zh, dtab_t = _stage_a1(h, W_h, W_attn)
    ae_t = _stage_a2(e, W_attn, W_e)
    gz, dd_t = _stage_g(zh, dtab_t, src, dst)
    y, ex_t = _stage_b(gz, ae_t, dd_t, W_attn)
    zeros_h = jnp.zeros((_NPAD, HD), jnp.float32)
    zeros_d = jnp.zeros((_NPAD,), jnp.float32)
    ph, *pds = _stage_s(y, ex_t, dst, zeros_h, zeros_d)
    pds2 = [jnp.reshape(p, (_NC, _NPAD)) for p in pds]
    h_out = _stage_d(ph, pds2, gamma_h, beta_h)
    return (h_out, e)


# speedup vs baseline: 28.5783x; 1.1187x over previous
import functools

import jax
import jax.numpy as jnp
from jax import lax
from jax.experimental import pallas as pl
from jax.experimental.pallas import tpu as pltpu
from jax.experimental.pallas import tpu_sc as plsc

N_NODES = 10000
N_EDGES = 320000
IN_DIM = 128
OUT_DIM = 32
N_HEADS = 4
HD = N_HEADS * OUT_DIM
EPS = 1e-5

_NC = 2
_NS = 16
_NW = _NC * _NS
_CH = 128
_SCE = 512
_NSC = N_EDGES // _SCE
_SPW = _NSC // _NW
_NREM = _NSC - _SPW * _NW
_SCS = 256
_NSCS = N_EDGES // _SCS
_SPWS = _NSCS // _NW
_NREMS = _NSCS - _SPWS * _NW
_NPAD = 10240
_SR = _NPAD // _NS

_BE = 12800
_NBE = N_EDGES // _BE
_BE2 = 12800
_NBE2 = N_EDGES // _BE2


def _stage_a1_body(h_ref, wh_ref, wa_ref, zh_ref, dt_ref):
    wa = wa_ref[...]
    h = h_ref[...]
    zs, drows = [], []
    for j in range(N_HEADS):
        z = jnp.dot(h, wh_ref[j])
        zs.append(z)
        wd = wa[j, 2 * OUT_DIM:3 * OUT_DIM][None, :]
        drows.append(lax.dot_general(wd, z, (((1,), (1,)), ((), ()))))
    zh_ref[...] = jnp.concatenate(zs, axis=1)
    dt_ref[...] = jnp.concatenate(drows, axis=0)


def _stage_a1(h, w_h, w_attn):
    return pl.pallas_call(
        _stage_a1_body,
        grid=(1,),
        in_specs=[
            pl.BlockSpec((N_NODES, IN_DIM), lambda i: (0, 0)),
            pl.BlockSpec((N_HEADS, IN_DIM, OUT_DIM), lambda i: (0, 0, 0)),
            pl.BlockSpec((N_HEADS, 3 * OUT_DIM), lambda i: (0, 0)),
        ],
        out_specs=[
            pl.BlockSpec((N_NODES, HD), lambda i: (0, 0)),
            pl.BlockSpec((N_HEADS, N_NODES), lambda i: (0, 0)),
        ],
        out_shape=[
            jax.ShapeDtypeStruct((N_NODES, HD), jnp.float32),
            jax.ShapeDtypeStruct((N_HEADS, N_NODES), jnp.float32),
        ],
    )(h, w_h, w_attn)


def _stage_a2_body(e_ref, wa_ref, we_ref, ae_ref):
    wa = wa_ref[...]
    ve_cols = [jnp.dot(we_ref[j], wa[j, 0:OUT_DIM][:, None])
               for j in range(N_HEADS)]
    v_e = jnp.concatenate(ve_cols, axis=1)
    ae_ref[...] = lax.dot_general(v_e, e_ref[...],
                                  (((0,), (1,)), ((), ())))


def _stage_a2(e, w_attn, w_e):
    return pl.pallas_call(
        _stage_a2_body,
        grid=(_NBE,),
        in_specs=[
            pl.BlockSpec((_BE, IN_DIM), lambda i: (i, 0)),
            pl.BlockSpec((N_HEADS, 3 * OUT_DIM), lambda i: (0, 0)),
            pl.BlockSpec((N_HEADS, IN_DIM, OUT_DIM), lambda i: (0, 0, 0)),
        ],
        out_specs=pl.BlockSpec((N_HEADS, _BE), lambda i: (0, i)),
        out_shape=jax.ShapeDtypeStruct((N_HEADS, N_EDGES), jnp.float32),
    )(e, w_attn, w_e)


def _gather_body(zh_hbm, dt0, dt1, dt2, dt3, src_hbm, dst_hbm,
                 gz_hbm, ddt_hbm,
                 sidx, didx, rows, ddbuf, sem, semd):
    wid = lax.axis_index("s") * _NC + lax.axis_index("c")
    dts = [dt0, dt1, dt2, dt3]

    def step(k, carry):
        off = (k * _NW + wid) * _SCE
        pltpu.sync_copy(src_hbm.at[pl.ds(off, _SCE)], sidx)
        pltpu.sync_copy(dst_hbm.at[pl.ds(off, _SCE)], didx)
        handles = [
            pltpu.async_copy(
                zh_hbm.at[sidx.at[pl.ds(t * _CH, _CH)]],
                rows.at[pl.ds(t * _CH, _CH)], sem)
            for t in range(_SCE // _CH)
        ]
        dhandles = [
            pltpu.async_copy(
                dts[j].at[didx.at[pl.ds(t * _CH, _CH)]],
                ddbuf.at[j, pl.ds(t * _CH, _CH)], semd)
            for t in range(_SCE // _CH)
            for j in range(N_HEADS)
        ]
        for hdl in handles:
            hdl.wait()
        pltpu.sync_copy(rows, gz_hbm.at[pl.ds(off, _SCE)])
        for hdl in dhandles:
            hdl.wait()
        pltpu.sync_copy(ddbuf, ddt_hbm.at[:, pl.ds(off, _SCE)])
        return carry

    niter = _SPW + jnp.where(wid < _NREM, 1, 0)
    lax.fori_loop(0, niter, step, 0)


def _stage_g(zh, dtab_t, src, dst):
    f = functools.partial(
        pl.kernel,
        out_type=(
            jax.ShapeDtypeStruct((N_EDGES, HD), jnp.float32),
            jax.ShapeDtypeStruct((N_HEADS, N_EDGES), jnp.float32),
        ),
        mesh=plsc.VectorSubcoreMesh(core_axis_name="c", subcore_axis_name="s"),
        scratch_types=[
            pltpu.VMEM((_SCE,), jnp.int32),
            pltpu.VMEM((_SCE,), jnp.int32),
            pltpu.VMEM((_SCE, HD), jnp.float32),
            pltpu.VMEM((N_HEADS, _SCE), jnp.float32),
            pltpu.SemaphoreType.DMA,
            pltpu.SemaphoreType.DMA,
        ],
    )(_gather_body)
    dts = [jnp.reshape(dtab_t[j], (N_NODES,)) for j in range(N_HEADS)]
    return f(zh, *dts, src, dst)


def _stage_b_body(gz_ref, ae_ref, dd_ref, wa_ref, y_ref, ex_ref):
    wa = wa_ref[...]
    gz = gz_ref[...]
    s_cols = [jnp.dot(gz[:, OUT_DIM * j:OUT_DIM * (j + 1)],
                      wa[j, OUT_DIM:2 * OUT_DIM][:, None])
              for j in range(N_HEADS)]
    eye4 = jnp.where(
        lax.broadcasted_iota(jnp.int32, (N_HEADS, N_HEADS), 0)
        == lax.broadcasted_iota(jnp.int32, (N_HEADS, N_HEADS), 1), 1.0, 0.0)
    add_t = ae_ref[...] + dd_ref[...]
    a = lax.dot_general(add_t, eye4, (((0,), (0,)), ((), ())))
    a = a + jnp.concatenate(s_cols, axis=1)
    a = jnp.where(a > 0, a, 0.01 * a)
    ex = jnp.exp(a)
    ex_ref[...] = lax.dot_general(eye4, ex, (((1,), (1,)), ((), ())))
    rep = jnp.where(
        lax.broadcasted_iota(jnp.int32, (N_HEADS, HD), 1) // OUT_DIM
        == lax.broadcasted_iota(jnp.int32, (N_HEADS, HD), 0), 1.0, 0.0)
    y_ref[...] = gz * jnp.dot(ex, rep)


def _stage_b(gz, ae_t, dd_t, w_attn):
    edge_spec = pl.BlockSpec((N_HEADS, _BE2), lambda i: (0, i))
    return pl.pallas_call(
        _stage_b_body,
        grid=(_NBE2,),
        in_specs=[
            pl.BlockSpec((_BE2, HD), lambda i: (i, 0)),
            edge_spec,
            edge_spec,
            pl.BlockSpec((N_HEADS, 3 * OUT_DIM), lambda i: (0, 0)),
        ],
        out_specs=[
            pl.BlockSpec((_BE2, HD), lambda i: (i, 0)),
            edge_spec,
        ],
        out_shape=[
            jax.ShapeDtypeStruct((N_EDGES, HD), jnp.float32),
            jax.ShapeDtypeStruct((N_HEADS, N_EDGES), jnp.float32),
        ],
    )(gz, ae_t, dd_t, w_attn)


def _scatter_body(y_hbm, ex_hbm, dst_hbm, zh_hbm, zd_hbm,
                  ph_hbm, pd0, pd1, pd2, pd3,
                  didx0, didx1, yrows, exbuf,
                  hsh, dsh0, dsh1, dsh2, dsh3, semi, sems, semd):
    cid = lax.axis_index("c")
    sid = lax.axis_index("s")
    wid = sid * _NC + cid
    didxs = [didx0, didx1]
    dshs = [dsh0, dsh1, dsh2, dsh3]
    pds = [pd0, pd1, pd2, pd3]

    pltpu.sync_copy(zh_hbm.at[pl.ds(sid * _SR, _SR)],
                    hsh.at[pl.ds(sid * _SR, _SR)])
    for j in range(N_HEADS):
        pltpu.sync_copy(zd_hbm.at[pl.ds(sid * _SR, _SR)],
                        dshs[j].at[pl.ds(sid * _SR, _SR)])
    plsc.subcore_barrier()

    def step(k, carry):
        off = (k * _NW + wid) * _SCS
        ih = [pltpu.async_copy(dst_hbm.at[pl.ds(off + t * _CH, _CH)],
                               didxs[t], semi)
              for t in range(_SCS // _CH)]
        pltpu.sync_copy(y_hbm.at[pl.ds(off, _SCS)], yrows)
        pltpu.sync_copy(ex_hbm.at[:, pl.ds(off, _SCS)], exbuf)
        for h in ih:
            h.wait()
        ah = []
        for t in range(_SCS // _CH):
            ah.append(pltpu.async_copy(
                yrows.at[pl.ds(t * _CH, _CH)], hsh.at[didxs[t]], sems,
                add=True))
            for j in range(N_HEADS):
                ah.append(pltpu.async_copy(
                    exbuf.at[j, pl.ds(t * _CH, _CH)], dshs[j].at[didxs[t]],
                    semd, add=True))
        for h in ah:
            h.wait()
        return carry

    niter = _SPWS + jnp.where(wid < _NREMS, 1, 0)
    lax.fori_loop(0, niter, step, 0)
    plsc.subcore_barrier()

    out_off = cid * _NPAD + sid * _SR
    pltpu.sync_copy(hsh.at[pl.ds(sid * _SR, _SR)],
                    ph_hbm.at[pl.ds(out_off, _SR)])
    for j in range(N_HEADS):
        pltpu.sync_copy(dshs[j].at[pl.ds(sid * _SR, _SR)],
                        pds[j].at[pl.ds(out_off, _SR)])


def _stage_s(y, ex_t, dst, zeros_h, zeros_d):
    f = functools.partial(
        pl.kernel,
        out_type=(
            jax.ShapeDtypeStruct((_NC * _NPAD, HD), jnp.float32),
        ) + tuple(jax.ShapeDtypeStruct((_NC * _NPAD,), jnp.float32)
                  for _ in range(N_HEADS)),
        mesh=plsc.VectorSubcoreMesh(core_axis_name="c", subcore_axis_name="s"),
        scratch_types=[
            pltpu.VMEM((_CH,), jnp.int32),
            pltpu.VMEM((_CH,), jnp.int32),
            pltpu.VMEM((_SCS, HD), jnp.float32),
            pltpu.VMEM((N_HEADS, _SCS), jnp.float32),
            pltpu.VMEM_SHARED((_NPAD, HD), jnp.float32),
        ] + [pltpu.VMEM_SHARED((_NPAD,), jnp.float32)] * N_HEADS
          + [pltpu.SemaphoreType.DMA] * 3,
    )(_scatter_body)
    return f(y, ex_t, dst, zeros_h, zeros_d)


def _stage_d_body(ph_ref, pd0, pd1, pd2, pd3, g_ref, b_ref, out_ref):
    hagg = (ph_ref[0:N_NODES, :]
            + ph_ref[_NPAD:_NPAD + N_NODES, :])
    acc = jnp.zeros((N_NODES, OUT_DIM), dtype=jnp.float32)
    inv_n = 1.0 / N_NODES
    ones11 = jnp.ones((1, 1), jnp.float32)
    for j, pd in enumerate((pd0, pd1, pd2, pd3)):
        den_row = pd[0:1, :] + pd[1:2, :]
        dj = lax.dot_general(den_row, ones11, (((0,), (0,)), ((), ())))
        dj = dj[0:N_NODES, :]
        x = hagg[:, OUT_DIM * j:OUT_DIM * (j + 1)]
        x = jnp.where(dj > 0, x / dj, 0.0)
        mu = jnp.sum(x, axis=0, keepdims=True) * inv_n
        xc = x - mu
        var = jnp.sum(xc * xc, axis=0, keepdims=True) * inv_n
        y = xc * lax.rsqrt(var + EPS) * g_ref[j][None, :] + b_ref[j][None, :]
        y = jnp.where(y > 0, y, jnp.exp(jnp.minimum(y, 0.0)) - 1.0)
        acc = acc + y
    out_ref[...] = acc


def _stage_d(ph, pds, gamma_h, beta_h):
    pd_spec = pl.BlockSpec((_NC, _NPAD), lambda i: (0, 0))
    return pl.pallas_call(
        _stage_d_body,
        grid=(1,),
        in_specs=[
            pl.BlockSpec((_NC * _NPAD, HD), lambda i: (0, 0)),
            pd_spec, pd_spec, pd_spec, pd_spec,
            pl.BlockSpec((N_HEADS, OUT_DIM), lambda i: (0, 0)),
            pl.BlockSpec((N_HEADS, OUT_DIM), lambda i: (0, 0)),
        ],
        out_specs=pl.BlockSpec((N_NODES, OUT_DIM), lambda i: (0, 0)),
        out_shape=jax.ShapeDtypeStruct((N_NODES, OUT_DIM), jnp.float32),
    )(ph, *pds, gamma_h, beta_h)


def kernel(h, e, edge_index, W_h, W_e, W_proj, b_proj, W_attn,
           gamma_h, beta_h, gamma_e, beta_e):
    src = edge_index[0].astype(jnp.int32)
    dst = edge_index[1].astype(jnp.int32)

    zh, dtab_t = _stage_a1(h, W_h, W_attn)
    ae_t = _stage_a2(e, W_attn, W_e)
    gz, dd_t = _stage_g(zh, dtab_t, src, dst)
    y, ex_t = _stage_b(gz, ae_t, dd_t, W_attn)
    zeros_h = jnp.zeros((_NPAD, HD), jnp.float32)
    zeros_d = jnp.zeros((_NPAD,), jnp.float32)
    ph, *pds = _stage_s(y, ex_t, dst, zeros_h, zeros_d)
    pds2 = [jnp.reshape(p, (_NC, _NPAD)) for p in pds]
    h_out = _stage_d(ph, pds2, gamma_h, beta_h)
    return (h_out, e)


# speedup vs baseline: 34.2352x; 1.1979x over previous
import functools

import jax
import jax.numpy as jnp
from jax import lax
from jax.experimental import pallas as pl
from jax.experimental.pallas import tpu as pltpu
from jax.experimental.pallas import tpu_sc as plsc

N_NODES = 10000
N_EDGES = 320000
IN_DIM = 128
OUT_DIM = 32
N_HEADS = 4
HD = N_HEADS * OUT_DIM
EPS = 1e-5

_NC = 2
_NS = 16
_NW = _NC * _NS
_EH = N_EDGES // 2
_CH = 128
_SCG = 256
_NSCH = _EH // _SCG
_SPWH = _NSCH // _NW
_REMH = _NSCH - _SPWH * _NW
_NPAD = 10240
_SR = _NPAD // _NS

_BE = 12800
_NBE = N_EDGES // _BE
_BEH = 6400
_NBEH = _EH // _BEH


def _stage_a1_body(h_ref, wh_ref, wa_ref, zh_ref, dt_ref):
    wa = wa_ref[...]
    h = h_ref[...]
    zs, drows = [], []
    for j in range(N_HEADS):
        z = jnp.dot(h, wh_ref[j])
        zs.append(z)
        wd = wa[j, 2 * OUT_DIM:3 * OUT_DIM][None, :]
        drows.append(lax.dot_general(wd, z, (((1,), (1,)), ((), ()))))
    zh_ref[...] = jnp.concatenate(zs, axis=1)
    dt_ref[...] = jnp.concatenate(drows, axis=0)


def _stage_a1(h, w_h, w_attn):
    return pl.pallas_call(
        _stage_a1_body,
        grid=(1,),
        in_specs=[
            pl.BlockSpec((N_NODES, IN_DIM), lambda i: (0, 0)),
            pl.BlockSpec((N_HEADS, IN_DIM, OUT_DIM), lambda i: (0, 0, 0)),
            pl.BlockSpec((N_HEADS, 3 * OUT_DIM), lambda i: (0, 0)),
        ],
        out_specs=[
            pl.BlockSpec((N_NODES, HD), lambda i: (0, 0)),
            pl.BlockSpec((N_HEADS, N_NODES), lambda i: (0, 0)),
        ],
        out_shape=[
            jax.ShapeDtypeStruct((N_NODES, HD), jnp.float32),
            jax.ShapeDtypeStruct((N_HEADS, N_NODES), jnp.float32),
        ],
    )(h, w_h, w_attn)


def _stage_a2_body(e_ref, wa_ref, we_ref, ae_ref):
    wa = wa_ref[...]
    ve_cols = [jnp.dot(we_ref[j], wa[j, 0:OUT_DIM][:, None])
               for j in range(N_HEADS)]
    v_e = jnp.concatenate(ve_cols, axis=1)
    ae_ref[...] = lax.dot_general(v_e, e_ref[...],
                                  (((0,), (1,)), ((), ())))


def _stage_a2(e, w_attn, w_e):
    return pl.pallas_call(
        _stage_a2_body,
        grid=(_NBE,),
        in_specs=[
            pl.BlockSpec((_BE, IN_DIM), lambda i: (i, 0)),
            pl.BlockSpec((N_HEADS, 3 * OUT_DIM), lambda i: (0, 0)),
            pl.BlockSpec((N_HEADS, IN_DIM, OUT_DIM), lambda i: (0, 0, 0)),
        ],
        out_specs=pl.BlockSpec((N_HEADS, _BE), lambda i: (0, i)),
        out_shape=jax.ShapeDtypeStruct((N_HEADS, N_EDGES), jnp.float32),
    )(e, w_attn, w_e)


def _gather_body(zh_hbm, dt0, dt1, dt2, dt3, src_hbm, dst_hbm,
                 gz_hbm, ddt_hbm,
                 sidx, didx, rows, ddbuf, sem, semd):
    wid = lax.axis_index("s") * _NC + lax.axis_index("c")
    dts = [dt0, dt1, dt2, dt3]

    def step(k, carry):
        off = (k * _NW + wid) * _SCG
        pltpu.sync_copy(src_hbm.at[pl.ds(off, _SCG)], sidx)
        pltpu.sync_copy(dst_hbm.at[pl.ds(off, _SCG)], didx)
        handles = [
            pltpu.async_copy(
                zh_hbm.at[sidx.at[pl.ds(t * _CH, _CH)]],
                rows.at[pl.ds(t * _CH, _CH)], sem)
            for t in range(_SCG // _CH)
        ]
        dhandles = [
            pltpu.async_copy(
                dts[j].at[didx.at[pl.ds(t * _CH, _CH)]],
                ddbuf.at[j, pl.ds(t * _CH, _CH)], semd)
            for t in range(_SCG // _CH)
            for j in range(N_HEADS)
        ]
        for hdl in handles:
            hdl.wait()
        pltpu.sync_copy(rows, gz_hbm.at[pl.ds(off, _SCG)])
        for hdl in dhandles:
            hdl.wait()
        pltpu.sync_copy(ddbuf, ddt_hbm.at[:, pl.ds(off, _SCG)])
        return carry

    niter = _SPWH + jnp.where(wid < _REMH, 1, 0)
    lax.fori_loop(0, niter, step, 0)


def _stage_g(zh, dtab_t, src, dst):
    f = functools.partial(
        pl.kernel,
        out_type=(
            jax.ShapeDtypeStruct((_EH, HD), jnp.float32),
            jax.ShapeDtypeStruct((N_HEADS, _EH), jnp.float32),
        ),
        mesh=plsc.VectorSubcoreMesh(core_axis_name="c", subcore_axis_name="s"),
        scratch_types=[
            pltpu.VMEM((_SCG,), jnp.int32),
            pltpu.VMEM((_SCG,), jnp.int32),
            pltpu.VMEM((_SCG, HD), jnp.float32),
            pltpu.VMEM((N_HEADS, _SCG), jnp.float32),
            pltpu.SemaphoreType.DMA,
            pltpu.SemaphoreType.DMA,
        ],
    )(_gather_body)
    dts = [jnp.reshape(dtab_t[j], (N_NODES,)) for j in range(N_HEADS)]
    return f(zh, *dts, src, dst)


def _stage_b_body(gz_ref, ae_ref, dd_ref, wa_ref, y_ref, ex_ref):
    wa = wa_ref[...]
    gz = gz_ref[...]
    s_cols = [jnp.dot(gz[:, OUT_DIM * j:OUT_DIM * (j + 1)],
                      wa[j, OUT_DIM:2 * OUT_DIM][:, None])
              for j in range(N_HEADS)]
    eye4 = jnp.where(
        lax.broadcasted_iota(jnp.int32, (N_HEADS, N_HEADS), 0)
        == lax.broadcasted_iota(jnp.int32, (N_HEADS, N_HEADS), 1), 1.0, 0.0)
    add_t = ae_ref[...] + dd_ref[...]
    a = lax.dot_general(add_t, eye4, (((0,), (0,)), ((), ())))
    a = a + jnp.concatenate(s_cols, axis=1)
    a = jnp.where(a > 0, a, 0.01 * a)
    ex = jnp.exp(a)
    ex_ref[...] = lax.dot_general(eye4, ex, (((1,), (1,)), ((), ())))
    rep = jnp.where(
        lax.broadcasted_iota(jnp.int32, (N_HEADS, HD), 1) // OUT_DIM
        == lax.broadcasted_iota(jnp.int32, (N_HEADS, HD), 0), 1.0, 0.0)
    y_ref[...] = gz * jnp.dot(ex, rep)


def _stage_b(gz, ae_t, dd_t, w_attn):
    edge_spec = pl.BlockSpec((N_HEADS, _BEH), lambda i: (0, i))
    return pl.pallas_call(
        _stage_b_body,
        grid=(_NBEH,),
        in_specs=[
            pl.BlockSpec((_BEH, HD), lambda i: (i, 0)),
            edge_spec,
            edge_spec,
            pl.BlockSpec((N_HEADS, 3 * OUT_DIM), lambda i: (0, 0)),
        ],
        out_specs=[
            pl.BlockSpec((_BEH, HD), lambda i: (i, 0)),
            edge_spec,
        ],
        out_shape=[
            jax.ShapeDtypeStruct((_EH, HD), jnp.float32),
            jax.ShapeDtypeStruct((N_HEADS, _EH), jnp.float32),
        ],
    )(gz, ae_t, dd_t, w_attn)


def _scatter_body(y_hbm, ex_hbm, dst_hbm, zh_hbm, zd_hbm,
                  ph_hbm, pd0, pd1, pd2, pd3,
                  didx0, didx1, yrows, exbuf,
                  hsh, dsh0, dsh1, dsh2, dsh3, semi, sems, semd):
    cid = lax.axis_index("c")
    sid = lax.axis_index("s")
    wid = sid * _NC + cid
    didxs = [didx0, didx1]
    dshs = [dsh0, dsh1, dsh2, dsh3]
    pds = [pd0, pd1, pd2, pd3]

    pltpu.sync_copy(zh_hbm.at[pl.ds(sid * _SR, _SR)],
                    hsh.at[pl.ds(sid * _SR, _SR)])
    for j in range(N_HEADS):
        pltpu.sync_copy(zd_hbm.at[pl.ds(sid * _SR, _SR)],
                        dshs[j].at[pl.ds(sid * _SR, _SR)])
    plsc.subcore_barrier()

    def step(k, carry):
        off = (k * _NW + wid) * _SCG
        ih = [pltpu.async_copy(dst_hbm.at[pl.ds(off + t * _CH, _CH)],
                               didxs[t], semi)
              for t in range(_SCG // _CH)]
        pltpu.sync_copy(y_hbm.at[pl.ds(off, _SCG)], yrows)
        pltpu.sync_copy(ex_hbm.at[:, pl.ds(off, _SCG)], exbuf)
        for h in ih:
            h.wait()
        ah = []
        for t in range(_SCG // _CH):
            ah.append(pltpu.async_copy(
                yrows.at[pl.ds(t * _CH, _CH)], hsh.at[didxs[t]], sems,
                add=True))
            for j in range(N_HEADS):
                ah.append(pltpu.async_copy(
                    exbuf.at[j, pl.ds(t * _CH, _CH)], dshs[j].at[didxs[t]],
                    semd, add=True))
        for h in ah:
            h.wait()
        return carry

    niter = _SPWH + jnp.where(wid < _REMH, 1, 0)
    lax.fori_loop(0, niter, step, 0)
    plsc.subcore_barrier()

    out_off = cid * _NPAD + sid * _SR
    pltpu.sync_copy(hsh.at[pl.ds(sid * _SR, _SR)],
                    ph_hbm.at[pl.ds(out_off, _SR)])
    for j in range(N_HEADS):
        pltpu.sync_copy(dshs[j].at[pl.ds(sid * _SR, _SR)],
                        pds[j].at[pl.ds(out_off, _SR)])


def _stage_s(y, ex_t, dst, zeros_h, zeros_d):
    f = functools.partial(
        pl.kernel,
        out_type=(
            jax.ShapeDtypeStruct((_NC * _NPAD, HD), jnp.float32),
        ) + tuple(jax.ShapeDtypeStruct((_NC * _NPAD,), jnp.float32)
                  for _ in range(N_HEADS)),
        mesh=plsc.VectorSubcoreMesh(core_axis_name="c", subcore_axis_name="s"),
        scratch_types=[
            pltpu.VMEM((_CH,), jnp.int32),
            pltpu.VMEM((_CH,), jnp.int32),
            pltpu.VMEM((_SCG, HD), jnp.float32),
            pltpu.VMEM((N_HEADS, _SCG), jnp.float32),
            pltpu.VMEM_SHARED((_NPAD, HD), jnp.float32),
        ] + [pltpu.VMEM_SHARED((_NPAD,), jnp.float32)] * N_HEADS
          + [pltpu.SemaphoreType.DMA] * 3,
    )(_scatter_body)
    return f(y, ex_t, dst, zeros_h, zeros_d)


def _stage_d_body(pha_ref, phb_ref, pa0, pa1, pa2, pa3, pb0, pb1, pb2, pb3,
                  g_ref, b_ref, out_ref):
    hagg = (pha_ref[0:N_NODES, :]
            + pha_ref[_NPAD:_NPAD + N_NODES, :]
            + phb_ref[0:N_NODES, :]
            + phb_ref[_NPAD:_NPAD + N_NODES, :])
    acc = jnp.zeros((N_NODES, OUT_DIM), dtype=jnp.float32)
    inv_n = 1.0 / N_NODES
    ones11 = jnp.ones((1, 1), jnp.float32)
    pds_a = (pa0, pa1, pa2, pa3)
    pds_b = (pb0, pb1, pb2, pb3)
    for j in range(N_HEADS):
        pda, pdb = pds_a[j], pds_b[j]
        den_row = pda[0:1, :] + pda[1:2, :] + pdb[0:1, :] + pdb[1:2, :]
        dj = lax.dot_general(den_row, ones11, (((0,), (0,)), ((), ())))
        dj = dj[0:N_NODES, :]
        x = hagg[:, OUT_DIM * j:OUT_DIM * (j + 1)]
        x = jnp.where(dj > 0, x / dj, 0.0)
        mu = jnp.sum(x, axis=0, keepdims=True) * inv_n
        xc = x - mu
        var = jnp.sum(xc * xc, axis=0, keepdims=True) * inv_n
        y = xc * lax.rsqrt(var + EPS) * g_ref[j][None, :] + b_ref[j][None, :]
        y = jnp.where(y > 0, y, jnp.exp(jnp.minimum(y, 0.0)) - 1.0)
        acc = acc + y
    out_ref[...] = acc


def _stage_d(pha, phb, pds_a, pds_b, gamma_h, beta_h):
    pd_spec = pl.BlockSpec((_NC, _NPAD), lambda i: (0, 0))
    ph_spec = pl.BlockSpec((_NC * _NPAD, HD), lambda i: (0, 0))
    return pl.pallas_call(
        _stage_d_body,
        grid=(1,),
        in_specs=[
            ph_spec, ph_spec,
            pd_spec, pd_spec, pd_spec, pd_spec,
            pd_spec, pd_spec, pd_spec, pd_spec,
            pl.BlockSpec((N_HEADS, OUT_DIM), lambda i: (0, 0)),
            pl.BlockSpec((N_HEADS, OUT_DIM), lambda i: (0, 0)),
        ],
        out_specs=pl.BlockSpec((N_NODES, OUT_DIM), lambda i: (0, 0)),
        out_shape=jax.ShapeDtypeStruct((N_NODES, OUT_DIM), jnp.float32),
    )(pha, phb, *pds_a, *pds_b, gamma_h, beta_h)


def kernel(h, e, edge_index, W_h, W_e, W_proj, b_proj, W_attn,
           gamma_h, beta_h, gamma_e, beta_e):
    src = edge_index[0].astype(jnp.int32)
    dst = edge_index[1].astype(jnp.int32)

    zh, dtab_t = _stage_a1(h, W_h, W_attn)
    ae_t = _stage_a2(e, W_attn, W_e)
    zeros_h = jnp.zeros((_NPAD, HD), jnp.float32)
    zeros_d = jnp.zeros((_NPAD,), jnp.float32)

    gz1, dd1 = _stage_g(zh, dtab_t, src[:_EH], dst[:_EH])
    gz2, dd2 = _stage_g(zh, dtab_t, src[_EH:], dst[_EH:])
    y1, ex1 = _stage_b(gz1, ae_t[:, :_EH], dd1, W_attn)
    ph1, *pds1 = _stage_s(y1, ex1, dst[:_EH], zeros_h, zeros_d)
    y2, ex2 = _stage_b(gz2, ae_t[:, _EH:], dd2, W_attn)
    ph2, *pds2 = _stage_s(y2, ex2, dst[_EH:], zeros_h, zeros_d)
    pda = [jnp.reshape(p, (_NC, _NPAD)) for p in pds1]
    pdb = [jnp.reshape(p, (_NC, _NPAD)) for p in pds2]
    h_out = _stage_d(ph1, ph2, pda, pdb, gamma_h, beta_h)
    return (h_out, e)


# speedup vs baseline: 34.2487x; 1.0004x over previous
import functools

import jax
import jax.numpy as jnp
from jax import lax
from jax.experimental import pallas as pl
from jax.experimental.pallas import tpu as pltpu
from jax.experimental.pallas import tpu_sc as plsc

N_NODES = 10000
N_EDGES = 320000
IN_DIM = 128
OUT_DIM = 32
N_HEADS = 4
HD = N_HEADS * OUT_DIM
EPS = 1e-5

_NC = 2
_NS = 16
_NW = _NC * _NS
_EH = N_EDGES // 2
_CH = 128
_SCG = 256
_NSCH = _EH // _SCG
_SPWH = _NSCH // _NW
_REMH = _NSCH - _SPWH * _NW
_NPAD = 10240
_SR = _NPAD // _NS

_BE = 16000
_NBE = N_EDGES // _BE
_BEH = 6400
_NBEH = _EH // _BEH


def _stage_a1_body(h_ref, wh_ref, wa_ref, zh_ref, dt_ref):
    wa = wa_ref[...]
    h = h_ref[...]
    zs, drows = [], []
    for j in range(N_HEADS):
        z = jnp.dot(h, wh_ref[j])
        zs.append(z)
        wd = wa[j, 2 * OUT_DIM:3 * OUT_DIM][None, :]
        drows.append(lax.dot_general(wd, z, (((1,), (1,)), ((), ()))))
    zh_ref[...] = jnp.concatenate(zs, axis=1)
    dt_ref[...] = jnp.concatenate(drows, axis=0)


def _stage_a1(h, w_h, w_attn):
    return pl.pallas_call(
        _stage_a1_body,
        grid=(1,),
        in_specs=[
            pl.BlockSpec((N_NODES, IN_DIM), lambda i: (0, 0)),
            pl.BlockSpec((N_HEADS, IN_DIM, OUT_DIM), lambda i: (0, 0, 0)),
            pl.BlockSpec((N_HEADS, 3 * OUT_DIM), lambda i: (0, 0)),
        ],
        out_specs=[
            pl.BlockSpec((N_NODES, HD), lambda i: (0, 0)),
            pl.BlockSpec((N_HEADS, N_NODES), lambda i: (0, 0)),
        ],
        out_shape=[
            jax.ShapeDtypeStruct((N_NODES, HD), jnp.float32),
            jax.ShapeDtypeStruct((N_HEADS, N_NODES), jnp.float32),
        ],
    )(h, w_h, w_attn)


def _stage_a2_body(e_ref, wa_ref, we_ref, ae_ref):
    wa = wa_ref[...]
    ve_cols = [jnp.dot(we_ref[j], wa[j, 0:OUT_DIM][:, None])
               for j in range(N_HEADS)]
    v_e = jnp.concatenate(ve_cols, axis=1)
    ae_ref[...] = lax.dot_general(v_e, e_ref[...],
                                  (((0,), (1,)), ((), ())))


def _stage_a2(e, w_attn, w_e):
    return pl.pallas_call(
        _stage_a2_body,
        grid=(_NBE,),
        in_specs=[
            pl.BlockSpec((_BE, IN_DIM), lambda i: (i, 0)),
            pl.BlockSpec((N_HEADS, 3 * OUT_DIM), lambda i: (0, 0)),
            pl.BlockSpec((N_HEADS, IN_DIM, OUT_DIM), lambda i: (0, 0, 0)),
        ],
        out_specs=pl.BlockSpec((N_HEADS, _BE), lambda i: (0, i)),
        out_shape=jax.ShapeDtypeStruct((N_HEADS, N_EDGES), jnp.float32),
    )(e, w_attn, w_e)


def _gather_body(zh_hbm, dt0, dt1, dt2, dt3, src_hbm, dst_hbm,
                 gz_hbm, ddt_hbm,
                 sidx, didx, rows, ddbuf, sem, semd):
    wid = lax.axis_index("s") * _NC + lax.axis_index("c")
    dts = [dt0, dt1, dt2, dt3]

    def step(k, carry):
        off = (k * _NW + wid) * _SCG
        pltpu.sync_copy(src_hbm.at[pl.ds(off, _SCG)], sidx)
        pltpu.sync_copy(dst_hbm.at[pl.ds(off, _SCG)], didx)
        handles = [
            pltpu.async_copy(
                zh_hbm.at[sidx.at[pl.ds(t * _CH, _CH)]],
                rows.at[pl.ds(t * _CH, _CH)], sem)
            for t in range(_SCG // _CH)
        ]
        dhandles = [
            pltpu.async_copy(
                dts[j].at[didx.at[pl.ds(t * _CH, _CH)]],
                ddbuf.at[j, pl.ds(t * _CH, _CH)], semd)
            for t in range(_SCG // _CH)
            for j in range(N_HEADS)
        ]
        for hdl in handles:
            hdl.wait()
        pltpu.sync_copy(rows, gz_hbm.at[pl.ds(off, _SCG)])
        for hdl in dhandles:
            hdl.wait()
        pltpu.sync_copy(ddbuf, ddt_hbm.at[:, pl.ds(off, _SCG)])
        return carry

    niter = _SPWH + jnp.where(wid < _REMH, 1, 0)
    lax.fori_loop(0, niter, step, 0)


def _stage_g(zh, dtab_t, src, dst):
    f = functools.partial(
        pl.kernel,
        out_type=(
            jax.ShapeDtypeStruct((_EH, HD), jnp.float32),
            jax.ShapeDtypeStruct((N_HEADS, _EH), jnp.float32),
        ),
        mesh=plsc.VectorSubcoreMesh(core_axis_name="c", subcore_axis_name="s"),
        scratch_types=[
            pltpu.VMEM((_SCG,), jnp.int32),
            pltpu.VMEM((_SCG,), jnp.int32),
            pltpu.VMEM((_SCG, HD), jnp.float32),
            pltpu.VMEM((N_HEADS, _SCG), jnp.float32),
            pltpu.SemaphoreType.DMA,
            pltpu.SemaphoreType.DMA,
        ],
    )(_gather_body)
    dts = [jnp.reshape(dtab_t[j], (N_NODES,)) for j in range(N_HEADS)]
    return f(zh, *dts, src, dst)


def _stage_b_body(gz_ref, ae_ref, dd_ref, wa_ref, y_ref, ex_ref):
    wa = wa_ref[...]
    gz = gz_ref[...]
    s_cols = [jnp.dot(gz[:, OUT_DIM * j:OUT_DIM * (j + 1)],
                      wa[j, OUT_DIM:2 * OUT_DIM][:, None])
              for j in range(N_HEADS)]
    eye4 = jnp.where(
        lax.broadcasted_iota(jnp.int32, (N_HEADS, N_HEADS), 0)
        == lax.broadcasted_iota(jnp.int32, (N_HEADS, N_HEADS), 1), 1.0, 0.0)
    add_t = ae_ref[...] + dd_ref[...]
    a = lax.dot_general(add_t, eye4, (((0,), (0,)), ((), ())))
    a = a + jnp.concatenate(s_cols, axis=1)
    a = jnp.where(a > 0, a, 0.01 * a)
    ex = jnp.exp(a)
    ex_ref[...] = lax.dot_general(eye4, ex, (((1,), (1,)), ((), ())))
    rep = jnp.where(
        lax.broadcasted_iota(jnp.int32, (N_HEADS, HD), 1) // OUT_DIM
        == lax.broadcasted_iota(jnp.int32, (N_HEADS, HD), 0), 1.0, 0.0)
    y_ref[...] = gz * jnp.dot(ex, rep)


def _stage_b(gz, ae_t, dd_t, w_attn):
    edge_spec = pl.BlockSpec((N_HEADS, _BEH), lambda i: (0, i))
    return pl.pallas_call(
        _stage_b_body,
        grid=(_NBEH,),
        in_specs=[
            pl.BlockSpec((_BEH, HD), lambda i: (i, 0)),
            edge_spec,
            edge_spec,
            pl.BlockSpec((N_HEADS, 3 * OUT_DIM), lambda i: (0, 0)),
        ],
        out_specs=[
            pl.BlockSpec((_BEH, HD), lambda i: (i, 0)),
            edge_spec,
        ],
        out_shape=[
            jax.ShapeDtypeStruct((_EH, HD), jnp.float32),
            jax.ShapeDtypeStruct((N_HEADS, _EH), jnp.float32),
        ],
    )(gz, ae_t, dd_t, w_attn)


def _scatter_body(y_hbm, ex_hbm, dst_hbm, zh_hbm, zd_hbm,
                  ph_hbm, pd0, pd1, pd2, pd3,
                  didx0, didx1, yrows, exbuf,
                  hsh, dsh0, dsh1, dsh2, dsh3, semi, sems, semd):
    cid = lax.axis_index("c")
    sid = lax.axis_index("s")
    wid = sid * _NC + cid
    didxs = [didx0, didx1]
    dshs = [dsh0, dsh1, dsh2, dsh3]
    pds = [pd0, pd1, pd2, pd3]

    pltpu.sync_copy(zh_hbm.at[pl.ds(sid * _SR, _SR)],
                    hsh.at[pl.ds(sid * _SR, _SR)])
    for j in range(N_HEADS):
        pltpu.sync_copy(zd_hbm.at[pl.ds(sid * _SR, _SR)],
                        dshs[j].at[pl.ds(sid * _SR, _SR)])
    plsc.subcore_barrier()

    def step(k, carry):
        off = (k * _NW + wid) * _SCG
        ih = [pltpu.async_copy(dst_hbm.at[pl.ds(off + t * _CH, _CH)],
                               didxs[t], semi)
              for t in range(_SCG // _CH)]
        pltpu.sync_copy(y_hbm.at[pl.ds(off, _SCG)], yrows)
        pltpu.sync_copy(ex_hbm.at[:, pl.ds(off, _SCG)], exbuf)
        for h in ih:
            h.wait()
        ah = []
        for t in range(_SCG // _CH):
            ah.append(pltpu.async_copy(
                yrows.at[pl.ds(t * _CH, _CH)], hsh.at[didxs[t]], sems,
                add=True))
            for j in range(N_HEADS):
                ah.append(pltpu.async_copy(
                    exbuf.at[j, pl.ds(t * _CH, _CH)], dshs[j].at[didxs[t]],
                    semd, add=True))
        for h in ah:
            h.wait()
        return carry

    niter = _SPWH + jnp.where(wid < _REMH, 1, 0)
    lax.fori_loop(0, niter, step, 0)
    plsc.subcore_barrier()

    out_off = cid * _NPAD + sid * _SR
    pltpu.sync_copy(hsh.at[pl.ds(sid * _SR, _SR)],
                    ph_hbm.at[pl.ds(out_off, _SR)])
    for j in range(N_HEADS):
        pltpu.sync_copy(dshs[j].at[pl.ds(sid * _SR, _SR)],
                        pds[j].at[pl.ds(out_off, _SR)])


def _stage_s(y, ex_t, dst, zeros_h, zeros_d):
    f = functools.partial(
        pl.kernel,
        out_type=(
            jax.ShapeDtypeStruct((_NC * _NPAD, HD), jnp.float32),
        ) + tuple(jax.ShapeDtypeStruct((_NC * _NPAD,), jnp.float32)
                  for _ in range(N_HEADS)),
        mesh=plsc.VectorSubcoreMesh(core_axis_name="c", subcore_axis_name="s"),
        scratch_types=[
            pltpu.VMEM((_CH,), jnp.int32),
            pltpu.VMEM((_CH,), jnp.int32),
            pltpu.VMEM((_SCG, HD), jnp.float32),
            pltpu.VMEM((N_HEADS, _SCG), jnp.float32),
            pltpu.VMEM_SHARED((_NPAD, HD), jnp.float32),
        ] + [pltpu.VMEM_SHARED((_NPAD,), jnp.float32)] * N_HEADS
          + [pltpu.SemaphoreType.DMA] * 3,
    )(_scatter_body)
    return f(y, ex_t, dst, zeros_h, zeros_d)


def _stage_d_body(pha_ref, phb_ref, pa0, pa1, pa2, pa3, pb0, pb1, pb2, pb3,
                  g_ref, b_ref, out_ref):
    hagg = (pha_ref[0:N_NODES, :]
            + pha_ref[_NPAD:_NPAD + N_NODES, :]
            + phb_ref[0:N_NODES, :]
            + phb_ref[_NPAD:_NPAD + N_NODES, :])
    acc = jnp.zeros((N_NODES, OUT_DIM), dtype=jnp.float32)
    inv_n = 1.0 / N_NODES
    ones11 = jnp.ones((1, 1), jnp.float32)
    pds_a = (pa0, pa1, pa2, pa3)
    pds_b = (pb0, pb1, pb2, pb3)
    for j in range(N_HEADS):
        pda, pdb = pds_a[j], pds_b[j]
        den_row = pda[0:1, :] + pda[1:2, :] + pdb[0:1, :] + pdb[1:2, :]
        dj = lax.dot_general(den_row, ones11, (((0,), (0,)), ((), ())))
        dj = dj[0:N_NODES, :]
        x = hagg[:, OUT_DIM * j:OUT_DIM * (j + 1)]
        x = jnp.where(dj > 0, x / dj, 0.0)
        mu = jnp.sum(x, axis=0, keepdims=True) * inv_n
        xc = x - mu
        var = jnp.sum(xc * xc, axis=0, keepdims=True) * inv_n
        y = xc * lax.rsqrt(var + EPS) * g_ref[j][None, :] + b_ref[j][None, :]
        y = jnp.where(y > 0, y, jnp.exp(jnp.minimum(y, 0.0)) - 1.0)
        acc = acc + y
    out_ref[...] = acc


def _stage_d(pha, phb, pds_a, pds_b, gamma_h, beta_h):
    pd_spec = pl.BlockSpec((_NC, _NPAD), lambda i: (0, 0))
    ph_spec = pl.BlockSpec((_NC * _NPAD, HD), lambda i: (0, 0))
    return pl.pallas_call(
        _stage_d_body,
        grid=(1,),
        in_specs=[
            ph_spec, ph_spec,
            pd_spec, pd_spec, pd_spec, pd_spec,
            pd_spec, pd_spec, pd_spec, pd_spec,
            pl.BlockSpec((N_HEADS, OUT_DIM), lambda i: (0, 0)),
            pl.BlockSpec((N_HEADS, OUT_DIM), lambda i: (0, 0)),
        ],
        out_specs=pl.BlockSpec((N_NODES, OUT_DIM), lambda i: (0, 0)),
        out_shape=jax.ShapeDtypeStruct((N_NODES, OUT_DIM), jnp.float32),
    )(pha, phb, *pds_a, *pds_b, gamma_h, beta_h)


def kernel(h, e, edge_index, W_h, W_e, W_proj, b_proj, W_attn,
           gamma_h, beta_h, gamma_e, beta_e):
    src = edge_index[0].astype(jnp.int32)
    dst = edge_index[1].astype(jnp.int32)

    zh, dtab_t = _stage_a1(h, W_h, W_attn)
    ae_t = _stage_a2(e, W_attn, W_e)
    zeros_h = jnp.zeros((_NPAD, HD), jnp.float32)
    zeros_d = jnp.zeros((_NPAD,), jnp.float32)

    gz1, dd1 = _stage_g(zh, dtab_t, src[:_EH], dst[:_EH])
    gz2, dd2 = _stage_g(zh, dtab_t, src[_EH:], dst[_EH:])
    y1, ex1 = _stage_b(gz1, ae_t[:, :_EH], dd1, W_attn)
    ph1, *pds1 = _stage_s(y1, ex1, dst[:_EH], zeros_h, zeros_d)
    y2, ex2 = _stage_b(gz2, ae_t[:, _EH:], dd2, W_attn)
    ph2, *pds2 = _stage_s(y2, ex2, dst[_EH:], zeros_h, zeros_d)
    pda = [jnp.reshape(p, (_NC, _NPAD)) for p in pds1]
    pdb = [jnp.reshape(p, (_NC, _NPAD)) for p in pds2]
    h_out = _stage_d(ph1, ph2, pda, pdb, gamma_h, beta_h)
    return (h_out, e)


# speedup vs baseline: 38.4422x; 1.1224x over previous
import functools

import jax
import jax.numpy as jnp
from jax import lax
from jax.experimental import pallas as pl
from jax.experimental.pallas import tpu as pltpu
from jax.experimental.pallas import tpu_sc as plsc

N_NODES = 10000
N_EDGES = 320000
IN_DIM = 128
OUT_DIM = 32
N_HEADS = 4
HD = N_HEADS * OUT_DIM
EPS = 1e-5

_NC = 2
_NS = 16
_NW = _NC * _NS
_EH = N_EDGES // 2
_CH = 128
_SCG = 256
_NSCH = _EH // _SCG
_SPWH = _NSCH // _NW
_REMH = _NSCH - _SPWH * _NW
_NPAD = 10240
_SR = _NPAD // _NS

_BE = 16000
_NBE = N_EDGES // _BE
_BEH = 6400
_NBEH = _EH // _BEH


def _stage_a1_body(h_ref, wh_ref, wa_ref, zh_ref, dt_ref):
    wa = wa_ref[...]
    h = h_ref[...]
    zs, drows = [], []
    for j in range(N_HEADS):
        z = jnp.dot(h, wh_ref[j])
        zs.append(z)
        wd = wa[j, 2 * OUT_DIM:3 * OUT_DIM][None, :]
        drows.append(lax.dot_general(wd, z, (((1,), (1,)), ((), ()))))
    zh_ref[...] = jnp.concatenate(zs, axis=1)
    dt_ref[...] = jnp.concatenate(drows, axis=0)


def _stage_a1(h, w_h, w_attn):
    return pl.pallas_call(
        _stage_a1_body,
        grid=(1,),
        in_specs=[
            pl.BlockSpec((N_NODES, IN_DIM), lambda i: (0, 0)),
            pl.BlockSpec((N_HEADS, IN_DIM, OUT_DIM), lambda i: (0, 0, 0)),
            pl.BlockSpec((N_HEADS, 3 * OUT_DIM), lambda i: (0, 0)),
        ],
        out_specs=[
            pl.BlockSpec((N_NODES, HD), lambda i: (0, 0)),
            pl.BlockSpec((N_HEADS, N_NODES), lambda i: (0, 0)),
        ],
        out_shape=[
            jax.ShapeDtypeStruct((N_NODES, HD), jnp.float32),
            jax.ShapeDtypeStruct((N_HEADS, N_NODES), jnp.float32),
        ],
    )(h, w_h, w_attn)


def _stage_a2_body(e_ref, wa_ref, we_ref, ae_ref):
    wa = wa_ref[...]
    ve_cols = [jnp.dot(we_ref[j], wa[j, 0:OUT_DIM][:, None])
               for j in range(N_HEADS)]
    v_e = jnp.concatenate(ve_cols, axis=1)
    ae_ref[...] = lax.dot_general(v_e, e_ref[...],
                                  (((0,), (1,)), ((), ())))


def _stage_a2(e, w_attn, w_e):
    return pl.pallas_call(
        _stage_a2_body,
        grid=(_NBE,),
        in_specs=[
            pl.BlockSpec((_BE, IN_DIM), lambda i: (i, 0)),
            pl.BlockSpec((N_HEADS, 3 * OUT_DIM), lambda i: (0, 0)),
            pl.BlockSpec((N_HEADS, IN_DIM, OUT_DIM), lambda i: (0, 0, 0)),
        ],
        out_specs=pl.BlockSpec((N_HEADS, _BE), lambda i: (0, i)),
        out_shape=jax.ShapeDtypeStruct((N_HEADS, N_EDGES), jnp.float32),
    )(e, w_attn, w_e)


def _gather_body(zh_hbm, dt0, dt1, dt2, dt3, src_hbm, dst_hbm,
                 gz_hbm, ddt_hbm,
                 sidx, didx, rows, ddbuf, sem, semd):
    wid = lax.axis_index("s") * _NC + lax.axis_index("c")
    dts = [dt0, dt1, dt2, dt3]

    def step(k, carry):
        off = (k * _NW + wid) * _SCG
        pltpu.sync_copy(src_hbm.at[pl.ds(off, _SCG)], sidx)
        pltpu.sync_copy(dst_hbm.at[pl.ds(off, _SCG)], didx)
        handles = [
            pltpu.async_copy(
                zh_hbm.at[sidx.at[pl.ds(t * _CH, _CH)]],
                rows.at[pl.ds(t * _CH, _CH)], sem)
            for t in range(_SCG // _CH)
        ]
        dhandles = [
            pltpu.async_copy(
                dts[j].at[didx.at[pl.ds(t * _CH, _CH)]],
                ddbuf.at[j, pl.ds(t * _CH, _CH)], semd)
            for t in range(_SCG // _CH)
            for j in range(N_HEADS)
        ]
        for hdl in handles:
            hdl.wait()
        pltpu.sync_copy(rows, gz_hbm.at[pl.ds(off, _SCG)])
        for hdl in dhandles:
            hdl.wait()
        pltpu.sync_copy(ddbuf, ddt_hbm.at[:, pl.ds(off, _SCG)])
        return carry

    niter = _SPWH + jnp.where(wid < _REMH, 1, 0)
    lax.fori_loop(0, niter, step, 0)


def _stage_g(zh, dtab_t, src, dst):
    f = functools.partial(
        pl.kernel,
        out_type=(
            jax.ShapeDtypeStruct((_EH, HD), jnp.float32),
            jax.ShapeDtypeStruct((N_HEADS, _EH), jnp.float32),
        ),
        mesh=plsc.VectorSubcoreMesh(core_axis_name="c", subcore_axis_name="s"),
        scratch_types=[
            pltpu.VMEM((_SCG,), jnp.int32),
            pltpu.VMEM((_SCG,), jnp.int32),
            pltpu.VMEM((_SCG, HD), jnp.float32),
            pltpu.VMEM((N_HEADS, _SCG), jnp.float32),
            pltpu.SemaphoreType.DMA,
            pltpu.SemaphoreType.DMA,
        ],
    )(_gather_body)
    dts = [jnp.reshape(dtab_t[j], (N_NODES,)) for j in range(N_HEADS)]
    return f(zh, *dts, src, dst)


def _stage_b_body(gz_ref, ae_ref, dd_ref, wa_ref, y_ref, ex_ref):
    wa = wa_ref[...]
    gz = gz_ref[...]
    rowhead = lax.broadcasted_iota(jnp.int32, (HD, N_HEADS), 0) // OUT_DIM
    colj = lax.broadcasted_iota(jnp.int32, (HD, N_HEADS), 1)
    d_idx = jnp.where(
        lax.broadcasted_iota(jnp.int32, (HD, OUT_DIM), 0) % OUT_DIM
        == lax.broadcasted_iota(jnp.int32, (HD, OUT_DIM), 1), 1.0, 0.0)
    wmid = wa[:, OUT_DIM:2 * OUT_DIM]
    m_full = lax.dot_general(d_idx, wmid, (((1,), (1,)), ((), ())))
    m_mat = jnp.where(rowhead == colj, m_full, 0.0)
    eye4 = jnp.where(
        lax.broadcasted_iota(jnp.int32, (N_HEADS, N_HEADS), 0)
        == lax.broadcasted_iota(jnp.int32, (N_HEADS, N_HEADS), 1), 1.0, 0.0)
    add_t = ae_ref[...] + dd_ref[...]
    a = lax.dot_general(add_t, eye4, (((0,), (0,)), ((), ())))
    a = a + jnp.dot(gz, m_mat)
    a = jnp.where(a > 0, a, 0.01 * a)
    ex = jnp.exp(a)
    ex_ref[...] = lax.dot_general(eye4, ex, (((1,), (1,)), ((), ())))
    rep = jnp.where(
        lax.broadcasted_iota(jnp.int32, (N_HEADS, HD), 1) // OUT_DIM
        == lax.broadcasted_iota(jnp.int32, (N_HEADS, HD), 0), 1.0, 0.0)
    y_ref[...] = gz * jnp.dot(ex, rep)


def _stage_b(gz, ae_t, dd_t, w_attn):
    edge_spec = pl.BlockSpec((N_HEADS, _BEH), lambda i: (0, i))
    return pl.pallas_call(
        _stage_b_body,
        grid=(_NBEH,),
        in_specs=[
            pl.BlockSpec((_BEH, HD), lambda i: (i, 0)),
            edge_spec,
            edge_spec,
            pl.BlockSpec((N_HEADS, 3 * OUT_DIM), lambda i: (0, 0)),
        ],
        out_specs=[
            pl.BlockSpec((_BEH, HD), lambda i: (i, 0)),
            edge_spec,
        ],
        out_shape=[
            jax.ShapeDtypeStruct((_EH, HD), jnp.float32),
            jax.ShapeDtypeStruct((N_HEADS, _EH), jnp.float32),
        ],
    )(gz, ae_t, dd_t, w_attn)


def _scatter_body(y_hbm, ex_hbm, dst_hbm, zh_hbm, zd_hbm,
                  ph_hbm, pd0, pd1, pd2, pd3,
                  didx0, didx1, yrows, exbuf,
                  hsh, dsh0, dsh1, dsh2, dsh3, semi, sems, semd):
    cid = lax.axis_index("c")
    sid = lax.axis_index("s")
    wid = sid * _NC + cid
    didxs = [didx0, didx1]
    dshs = [dsh0, dsh1, dsh2, dsh3]
    pds = [pd0, pd1, pd2, pd3]

    pltpu.sync_copy(zh_hbm.at[pl.ds(sid * _SR, _SR)],
                    hsh.at[pl.ds(sid * _SR, _SR)])
    for j in range(N_HEADS):
        pltpu.sync_copy(zd_hbm.at[pl.ds(sid * _SR, _SR)],
                        dshs[j].at[pl.ds(sid * _SR, _SR)])
    plsc.subcore_barrier()

    def step(k, carry):
        off = (k * _NW + wid) * _SCG
        ih = [pltpu.async_copy(dst_hbm.at[pl.ds(off + t * _CH, _CH)],
                               didxs[t], semi)
              for t in range(_SCG // _CH)]
        pltpu.sync_copy(y_hbm.at[pl.ds(off, _SCG)], yrows)
        pltpu.sync_copy(ex_hbm.at[:, pl.ds(off, _SCG)], exbuf)
        for h in ih:
            h.wait()
        ah = []
        for t in range(_SCG // _CH):
            ah.append(pltpu.async_copy(
                yrows.at[pl.ds(t * _CH, _CH)], hsh.at[didxs[t]], sems,
                add=True))
            for j in range(N_HEADS):
                ah.append(pltpu.async_copy(
                    exbuf.at[j, pl.ds(t * _CH, _CH)], dshs[j].at[didxs[t]],
                    semd, add=True))
        for h in ah:
            h.wait()
        return carry

    niter = _SPWH + jnp.where(wid < _REMH, 1, 0)
    lax.fori_loop(0, niter, step, 0)
    plsc.subcore_barrier()

    out_off = cid * _NPAD + sid * _SR
    pltpu.sync_copy(hsh.at[pl.ds(sid * _SR, _SR)],
                    ph_hbm.at[pl.ds(out_off, _SR)])
    for j in range(N_HEADS):
        pltpu.sync_copy(dshs[j].at[pl.ds(sid * _SR, _SR)],
                        pds[j].at[pl.ds(out_off, _SR)])


def _stage_s(y, ex_t, dst, zeros_h, zeros_d):
    f = functools.partial(
        pl.kernel,
        out_type=(
            jax.ShapeDtypeStruct((_NC * _NPAD, HD), jnp.float32),
        ) + tuple(jax.ShapeDtypeStruct((_NC * _NPAD,), jnp.float32)
                  for _ in range(N_HEADS)),
        mesh=plsc.VectorSubcoreMesh(core_axis_name="c", subcore_axis_name="s"),
        scratch_types=[
            pltpu.VMEM((_CH,), jnp.int32),
            pltpu.VMEM((_CH,), jnp.int32),
            pltpu.VMEM((_SCG, HD), jnp.float32),
            pltpu.VMEM((N_HEADS, _SCG), jnp.float32),
            pltpu.VMEM_SHARED((_NPAD, HD), jnp.float32),
        ] + [pltpu.VMEM_SHARED((_NPAD,), jnp.float32)] * N_HEADS
          + [pltpu.SemaphoreType.DMA] * 3,
    )(_scatter_body)
    return f(y, ex_t, dst, zeros_h, zeros_d)


def _stage_d_body(pha_ref, phb_ref, pa0, pa1, pa2, pa3, pb0, pb1, pb2, pb3,
                  g_ref, b_ref, out_ref):
    hagg = (pha_ref[0:N_NODES, :]
            + pha_ref[_NPAD:_NPAD + N_NODES, :]
            + phb_ref[0:N_NODES, :]
            + phb_ref[_NPAD:_NPAD + N_NODES, :])
    acc = jnp.zeros((N_NODES, OUT_DIM), dtype=jnp.float32)
    inv_n = 1.0 / N_NODES
    ones11 = jnp.ones((1, 1), jnp.float32)
    pds_a = (pa0, pa1, pa2, pa3)
    pds_b = (pb0, pb1, pb2, pb3)
    for j in range(N_HEADS):
        pda, pdb = pds_a[j], pds_b[j]
        den_row = pda[0:1, :] + pda[1:2, :] + pdb[0:1, :] + pdb[1:2, :]
        dj = lax.dot_general(den_row, ones11, (((0,), (0,)), ((), ())))
        dj = dj[0:N_NODES, :]
        x = hagg[:, OUT_DIM * j:OUT_DIM * (j + 1)]
        x = jnp.where(dj > 0, x / dj, 0.0)
        mu = jnp.sum(x, axis=0, keepdims=True) * inv_n
        xc = x - mu
        var = jnp.sum(xc * xc, axis=0, keepdims=True) * inv_n
        y = xc * lax.rsqrt(var + EPS) * g_ref[j][None, :] + b_ref[j][None, :]
        y = jnp.where(y > 0, y, jnp.exp(jnp.minimum(y, 0.0)) - 1.0)
        acc = acc + y
    out_ref[...] = acc


def _stage_d(pha, phb, pds_a, pds_b, gamma_h, beta_h):
    pd_spec = pl.BlockSpec((_NC, _NPAD), lambda i: (0, 0))
    ph_spec = pl.BlockSpec((_NC * _NPAD, HD), lambda i: (0, 0))
    return pl.pallas_call(
        _stage_d_body,
        grid=(1,),
        in_specs=[
            ph_spec, ph_spec,
            pd_spec, pd_spec, pd_spec, pd_spec,
            pd_spec, pd_spec, pd_spec, pd_spec,
            pl.BlockSpec((N_HEADS, OUT_DIM), lambda i: (0, 0)),
            pl.BlockSpec((N_HEADS, OUT_DIM), lambda i: (0, 0)),
        ],
        out_specs=pl.BlockSpec((N_NODES, OUT_DIM), lambda i: (0, 0)),
        out_shape=jax.ShapeDtypeStruct((N_NODES, OUT_DIM), jnp.float32),
    )(pha, phb, *pds_a, *pds_b, gamma_h, beta_h)


def kernel(h, e, edge_index, W_h, W_e, W_proj, b_proj, W_attn,
           gamma_h, beta_h, gamma_e, beta_e):
    src = edge_index[0].astype(jnp.int32)
    dst = edge_index[1].astype(jnp.int32)

    zh, dtab_t = _stage_a1(h, W_h, W_attn)
    ae_t = _stage_a2(e, W_attn, W_e)
    zeros_h = jnp.zeros((_NPAD, HD), jnp.float32)
    zeros_d = jnp.zeros((_NPAD,), jnp.float32)

    gz1, dd1 = _stage_g(zh, dtab_t, src[:_EH], dst[:_EH])
    gz2, dd2 = _stage_g(zh, dtab_t, src[_EH:], dst[_EH:])
    y1, ex1 = _stage_b(gz1, ae_t[:, :_EH], dd1, W_attn)
    ph1, *pds1 = _stage_s(y1, ex1, dst[:_EH], zeros_h, zeros_d)
    y2, ex2 = _stage_b(gz2, ae_t[:, _EH:], dd2, W_attn)
    ph2, *pds2 = _stage_s(y2, ex2, dst[_EH:], zeros_h, zeros_d)
    pda = [jnp.reshape(p, (_NC, _NPAD)) for p in pds1]
    pdb = [jnp.reshape(p, (_NC, _NPAD)) for p in pds2]
    h_out = _stage_d(ph1, ph2, pda, pdb, gamma_h, beta_h)
    return (h_out, e)


# speedup vs baseline: 38.9635x; 1.0136x over previous
import functools

import jax
import jax.numpy as jnp
from jax import lax
from jax.experimental import pallas as pl
from jax.experimental.pallas import tpu as pltpu
from jax.experimental.pallas import tpu_sc as plsc

N_NODES = 10000
N_EDGES = 320000
IN_DIM = 128
OUT_DIM = 32
N_HEADS = 4
HD = N_HEADS * OUT_DIM
EPS = 1e-5

_NC = 2
_NS = 16
_NW = _NC * _NS
_EH1 = 163840
_EH2 = N_EDGES - _EH1
_CH = 128
_SCG = 512
_SCS = 256
_NPAD = 10240
_SR = _NPAD // _NS

_BE = 16000
_NBE = N_EDGES // _BE
_BH1 = 8192
_BH2 = 7808


def _stage_a1_body(h_ref, wh_ref, wa_ref, zh_ref, dt_ref):
    wa = wa_ref[...]
    h = h_ref[...]
    zs, drows = [], []
    for j in range(N_HEADS):
        z = jnp.dot(h, wh_ref[j])
        zs.append(z)
        wd = wa[j, 2 * OUT_DIM:3 * OUT_DIM][None, :]
        drows.append(lax.dot_general(wd, z, (((1,), (1,)), ((), ()))))
    zh_ref[...] = jnp.concatenate(zs, axis=1)
    dt_ref[...] = jnp.concatenate(drows, axis=0)


def _stage_a1(h, w_h, w_attn):
    return pl.pallas_call(
        _stage_a1_body,
        grid=(1,),
        in_specs=[
            pl.BlockSpec((N_NODES, IN_DIM), lambda i: (0, 0)),
            pl.BlockSpec((N_HEADS, IN_DIM, OUT_DIM), lambda i: (0, 0, 0)),
            pl.BlockSpec((N_HEADS, 3 * OUT_DIM), lambda i: (0, 0)),
        ],
        out_specs=[
            pl.BlockSpec((N_NODES, HD), lambda i: (0, 0)),
            pl.BlockSpec((N_HEADS, N_NODES), lambda i: (0, 0)),
        ],
        out_shape=[
            jax.ShapeDtypeStruct((N_NODES, HD), jnp.float32),
            jax.ShapeDtypeStruct((N_HEADS, N_NODES), jnp.float32),
        ],
    )(h, w_h, w_attn)


def _stage_a2_body(e_ref, wa_ref, we_ref, ae_ref):
    wa = wa_ref[...]
    ve_cols = [jnp.dot(we_ref[j], wa[j, 0:OUT_DIM][:, None])
               for j in range(N_HEADS)]
    v_e = jnp.concatenate(ve_cols, axis=1)
    ae_ref[...] = lax.dot_general(v_e, e_ref[...],
                                  (((0,), (1,)), ((), ())))


def _stage_a2(e, w_attn, w_e):
    return pl.pallas_call(
        _stage_a2_body,
        grid=(_NBE,),
        in_specs=[
            pl.BlockSpec((_BE, IN_DIM), lambda i: (i, 0)),
            pl.BlockSpec((N_HEADS, 3 * OUT_DIM), lambda i: (0, 0)),
            pl.BlockSpec((N_HEADS, IN_DIM, OUT_DIM), lambda i: (0, 0, 0)),
        ],
        out_specs=pl.BlockSpec((N_HEADS, _BE), lambda i: (0, i)),
        out_shape=jax.ShapeDtypeStruct((N_HEADS, N_EDGES), jnp.float32),
    )(e, w_attn, w_e)


def _make_gather_body(spw, rem):
  def _gather_body(zh_hbm, dt0, dt1, dt2, dt3, src_hbm, dst_hbm,
                 gz_hbm, ddt_hbm,
                 sidx, didx, rows, ddbuf, sem, semd):
    wid = lax.axis_index("s") * _NC + lax.axis_index("c")
    dts = [dt0, dt1, dt2, dt3]

    def step(k, carry):
        off = (k * _NW + wid) * _SCG
        pltpu.sync_copy(src_hbm.at[pl.ds(off, _SCG)], sidx)
        pltpu.sync_copy(dst_hbm.at[pl.ds(off, _SCG)], didx)
        handles = [
            pltpu.async_copy(
                zh_hbm.at[sidx.at[pl.ds(t * _CH, _CH)]],
                rows.at[pl.ds(t * _CH, _CH)], sem)
            for t in range(_SCG // _CH)
        ]
        dhandles = [
            pltpu.async_copy(
                dts[j].at[didx.at[pl.ds(t * _CH, _CH)]],
                ddbuf.at[j, pl.ds(t * _CH, _CH)], semd)
            for t in range(_SCG // _CH)
            for j in range(N_HEADS)
        ]
        for hdl in handles:
            hdl.wait()
        pltpu.sync_copy(rows, gz_hbm.at[pl.ds(off, _SCG)])
        for hdl in dhandles:
            hdl.wait()
        pltpu.sync_copy(ddbuf, ddt_hbm.at[:, pl.ds(off, _SCG)])
        return carry

    niter = spw + jnp.where(wid < rem, 1, 0)
    lax.fori_loop(0, niter, step, 0)
  return _gather_body


def _stage_g(zh, dtab_t, src, dst):
    n = src.shape[0]
    nsc = n // _SCG
    f = functools.partial(
        pl.kernel,
        out_type=(
            jax.ShapeDtypeStruct((n, HD), jnp.float32),
            jax.ShapeDtypeStruct((N_HEADS, n), jnp.float32),
        ),
        mesh=plsc.VectorSubcoreMesh(core_axis_name="c", subcore_axis_name="s"),
        scratch_types=[
            pltpu.VMEM((_SCG,), jnp.int32),
            pltpu.VMEM((_SCG,), jnp.int32),
            pltpu.VMEM((_SCG, HD), jnp.float32),
            pltpu.VMEM((N_HEADS, _SCG), jnp.float32),
            pltpu.SemaphoreType.DMA,
            pltpu.SemaphoreType.DMA,
        ],
    )(_make_gather_body(nsc // _NW, nsc % _NW))
    dts = [jnp.reshape(dtab_t[j], (N_NODES,)) for j in range(N_HEADS)]
    return f(zh, *dts, src, dst)


def _stage_b_body(gz_ref, ae_ref, dd_ref, wa_ref, y_ref, ex_ref):
    wa = wa_ref[...]
    gz = gz_ref[...]
    rowhead = lax.broadcasted_iota(jnp.int32, (HD, N_HEADS), 0) // OUT_DIM
    colj = lax.broadcasted_iota(jnp.int32, (HD, N_HEADS), 1)
    d_idx = jnp.where(
        lax.broadcasted_iota(jnp.int32, (HD, OUT_DIM), 0) % OUT_DIM
        == lax.broadcasted_iota(jnp.int32, (HD, OUT_DIM), 1), 1.0, 0.0)
    wmid = wa[:, OUT_DIM:2 * OUT_DIM]
    m_full = lax.dot_general(d_idx, wmid, (((1,), (1,)), ((), ())))
    m_mat = jnp.where(rowhead == colj, m_full, 0.0)
    eye4 = jnp.where(
        lax.broadcasted_iota(jnp.int32, (N_HEADS, N_HEADS), 0)
        == lax.broadcasted_iota(jnp.int32, (N_HEADS, N_HEADS), 1), 1.0, 0.0)
    add_t = ae_ref[...] + dd_ref[...]
    a = lax.dot_general(add_t, eye4, (((0,), (0,)), ((), ())))
    a = a + jnp.dot(gz, m_mat)
    a = jnp.where(a > 0, a, 0.01 * a)
    ex = jnp.exp(a)
    ex_ref[...] = lax.dot_general(eye4, ex, (((1,), (1,)), ((), ())))
    rep = jnp.where(
        lax.broadcasted_iota(jnp.int32, (N_HEADS, HD), 1) // OUT_DIM
        == lax.broadcasted_iota(jnp.int32, (N_HEADS, HD), 0), 1.0, 0.0)
    y_ref[...] = gz * jnp.dot(ex, rep)


def _stage_b(gz, ae_t, dd_t, w_attn, block):
    n = gz.shape[0]
    edge_spec = pl.BlockSpec((N_HEADS, block), lambda i: (0, i))
    return pl.pallas_call(
        _stage_b_body,
        grid=(n // block,),
        in_specs=[
            pl.BlockSpec((block, HD), lambda i: (i, 0)),
            edge_spec,
            edge_spec,
            pl.BlockSpec((N_HEADS, 3 * OUT_DIM), lambda i: (0, 0)),
        ],
        out_specs=[
            pl.BlockSpec((block, HD), lambda i: (i, 0)),
            edge_spec,
        ],
        out_shape=[
            jax.ShapeDtypeStruct((n, HD), jnp.float32),
            jax.ShapeDtypeStruct((N_HEADS, n), jnp.float32),
        ],
    )(gz, ae_t, dd_t, w_attn)


def _make_scatter_body(spw, rem):
  def _scatter_body(y_hbm, ex_hbm, dst_hbm, zh_hbm, zd_hbm,
                  ph_hbm, pd0, pd1, pd2, pd3,
                  didx0, didx1, yrows, exbuf,
                  hsh, dsh0, dsh1, dsh2, dsh3, semi, sems, semd):
    cid = lax.axis_index("c")
    sid = lax.axis_index("s")
    wid = sid * _NC + cid
    didxs = [didx0, didx1]
    dshs = [dsh0, dsh1, dsh2, dsh3]
    pds = [pd0, pd1, pd2, pd3]

    pltpu.sync_copy(zh_hbm.at[pl.ds(sid * _SR, _SR)],
                    hsh.at[pl.ds(sid * _SR, _SR)])
    for j in range(N_HEADS):
        pltpu.sync_copy(zd_hbm.at[pl.ds(sid * _SR, _SR)],
                        dshs[j].at[pl.ds(sid * _SR, _SR)])
    plsc.subcore_barrier()

    def step(k, carry):
        off = (k * _NW + wid) * _SCS
        ih = [pltpu.async_copy(dst_hbm.at[pl.ds(off + t * _CH, _CH)],
                               didxs[t], semi)
              for t in range(_SCS // _CH)]
        pltpu.sync_copy(y_hbm.at[pl.ds(off, _SCS)], yrows)
        pltpu.sync_copy(ex_hbm.at[:, pl.ds(off, _SCS)], exbuf)
        for h in ih:
            h.wait()
        ah = []
        for t in range(_SCS // _CH):
            ah.append(pltpu.async_copy(
                yrows.at[pl.ds(t * _CH, _CH)], hsh.at[didxs[t]], sems,
                add=True))
            for j in range(N_HEADS):
                ah.append(pltpu.async_copy(
                    exbuf.at[j, pl.ds(t * _CH, _CH)], dshs[j].at[didxs[t]],
                    semd, add=True))
        for h in ah:
            h.wait()
        return carry

    niter = spw + jnp.where(wid < rem, 1, 0)
    lax.fori_loop(0, niter, step, 0)
    plsc.subcore_barrier()

    out_off = cid * _NPAD + sid * _SR
    pltpu.sync_copy(hsh.at[pl.ds(sid * _SR, _SR)],
                    ph_hbm.at[pl.ds(out_off, _SR)])
    for j in range(N_HEADS):
        pltpu.sync_copy(dshs[j].at[pl.ds(sid * _SR, _SR)],
                        pds[j].at[pl.ds(out_off, _SR)])
  return _scatter_body


def _stage_s(y, ex_t, dst, zeros_h, zeros_d):
    nsc = y.shape[0] // _SCS
    f = functools.partial(
        pl.kernel,
        out_type=(
            jax.ShapeDtypeStruct((_NC * _NPAD, HD), jnp.float32),
        ) + tuple(jax.ShapeDtypeStruct((_NC * _NPAD,), jnp.float32)
                  for _ in range(N_HEADS)),
        mesh=plsc.VectorSubcoreMesh(core_axis_name="c", subcore_axis_name="s"),
        scratch_types=[
            pltpu.VMEM((_CH,), jnp.int32),
            pltpu.VMEM((_CH,), jnp.int32),
            pltpu.VMEM((_SCS, HD), jnp.float32),
            pltpu.VMEM((N_HEADS, _SCS), jnp.float32),
            pltpu.VMEM_SHARED((_NPAD, HD), jnp.float32),
        ] + [pltpu.VMEM_SHARED((_NPAD,), jnp.float32)] * N_HEADS
          + [pltpu.SemaphoreType.DMA] * 3,
    )(_make_scatter_body(nsc // _NW, nsc % _NW))
    return f(y, ex_t, dst, zeros_h, zeros_d)


def _stage_d_body(pha_ref, phb_ref, pa0, pa1, pa2, pa3, pb0, pb1, pb2, pb3,
                  g_ref, b_ref, out_ref):
    hagg = (pha_ref[0:N_NODES, :]
            + pha_ref[_NPAD:_NPAD + N_NODES, :]
            + phb_ref[0:N_NODES, :]
            + phb_ref[_NPAD:_NPAD + N_NODES, :])
    acc = jnp.zeros((N_NODES, OUT_DIM), dtype=jnp.float32)
    inv_n = 1.0 / N_NODES
    ones11 = jnp.ones((1, 1), jnp.float32)
    pds_a = (pa0, pa1, pa2, pa3)
    pds_b = (pb0, pb1, pb2, pb3)
    for j in range(N_HEADS):
        pda, pdb = pds_a[j], pds_b[j]
        den_row = pda[0:1, :] + pda[1:2, :] + pdb[0:1, :] + pdb[1:2, :]
        dj = lax.dot_general(den_row, ones11, (((0,), (0,)), ((), ())))
        dj = dj[0:N_NODES, :]
        x = hagg[:, OUT_DIM * j:OUT_DIM * (j + 1)]
        x = jnp.where(dj > 0, x / dj, 0.0)
        mu = jnp.sum(x, axis=0, keepdims=True) * inv_n
        xc = x - mu
        var = jnp.sum(xc * xc, axis=0, keepdims=True) * inv_n
        y = xc * lax.rsqrt(var + EPS) * g_ref[j][None, :] + b_ref[j][None, :]
        y = jnp.where(y > 0, y, jnp.exp(jnp.minimum(y, 0.0)) - 1.0)
        acc = acc + y
    out_ref[...] = acc


def _stage_d(pha, phb, pds_a, pds_b, gamma_h, beta_h):
    pd_spec = pl.BlockSpec((_NC, _NPAD), lambda i: (0, 0))
    ph_spec = pl.BlockSpec((_NC * _NPAD, HD), lambda i: (0, 0))
    return pl.pallas_call(
        _stage_d_body,
        grid=(1,),
        in_specs=[
            ph_spec, ph_spec,
            pd_spec, pd_spec, pd_spec, pd_spec,
            pd_spec, pd_spec, pd_spec, pd_spec,
            pl.BlockSpec((N_HEADS, OUT_DIM), lambda i: (0, 0)),
            pl.BlockSpec((N_HEADS, OUT_DIM), lambda i: (0, 0)),
        ],
        out_specs=pl.BlockSpec((N_NODES, OUT_DIM), lambda i: (0, 0)),
        out_shape=jax.ShapeDtypeStruct((N_NODES, OUT_DIM), jnp.float32),
    )(pha, phb, *pds_a, *pds_b, gamma_h, beta_h)


def kernel(h, e, edge_index, W_h, W_e, W_proj, b_proj, W_attn,
           gamma_h, beta_h, gamma_e, beta_e):
    src = edge_index[0].astype(jnp.int32)
    dst = edge_index[1].astype(jnp.int32)

    zh, dtab_t = _stage_a1(h, W_h, W_attn)
    ae_t = _stage_a2(e, W_attn, W_e)
    zeros_h = jnp.zeros((_NPAD, HD), jnp.float32)
    zeros_d = jnp.zeros((_NPAD,), jnp.float32)

    gz1, dd1 = _stage_g(zh, dtab_t, src[:_EH1], dst[:_EH1])
    gz2, dd2 = _stage_g(zh, dtab_t, src[_EH1:], dst[_EH1:])
    y1, ex1 = _stage_b(gz1, ae_t[:, :_EH1], dd1, W_attn, _BH1)
    ph1, *pds1 = _stage_s(y1, ex1, dst[:_EH1], zeros_h, zeros_d)
    y2, ex2 = _stage_b(gz2, ae_t[:, _EH1:], dd2, W_attn, _BH2)
    ph2, *pds2 = _stage_s(y2, ex2, dst[_EH1:], zeros_h, zeros_d)
    pda = [jnp.reshape(p, (_NC, _NPAD)) for p in pds1]
    pdb = [jnp.reshape(p, (_NC, _NPAD)) for p in pds2]
    h_out = _stage_d(ph1, ph2, pda, pdb, gamma_h, beta_h)
    return (h_out, e)
